# Initial kernel scaffold; baseline (speedup 1.0000x reference)
import functools

import jax
import jax.numpy as jnp
from jax import lax
from jax.experimental import pallas as pl
from jax.experimental.pallas import tpu as pltpu
from jax.experimental.pallas import tpu_sc as plsc

N = 10000
E = 160000
D_IN, D_HID, D_OUT = 256, 128, 16
MU = 0.1
SCALING = 2.0

NC, NS, LL = 2, 16, 16
NW = NC * NS
N_PAD = 10240
E_PAD = 163840
EPT = E_PAD // NW
ECH = 128
NCH = EPT // ECH
NPT = N_PAD // NS

_P_HIGH = lax.Precision.HIGHEST


def _dot(a, b):
    return lax.dot_general(a, b, (((1,), (0,)), ((), ())),
                           precision=_P_HIGH, preferred_element_type=jnp.float32)


def _front_body(x_ref, wp1, bp1, wp2, bp2, wp3, bp3, pars, w1, b1, wc0, bc0,
                logits_ref, q_ref, f0_ref):
    xb = x_ref[...]
    hp = jnp.maximum(_dot(xb, wp1[...]) + bp1[...][None, :], 0.0)
    hp = jnp.maximum(_dot(hp, wp2[...]) + bp2[...][None, :], 0.0)
    logits = _dot(hp, wp3[...]) + bp3[...][None, :]
    logits_ref[...] = logits
    parsing = jnp.maximum(SCALING * pars[...], 0.0)
    q_ref[...] = _dot(logits, parsing)
    h1 = _dot(xb, w1[...]) + b1[...][None, :]
    f0_ref[...] = _dot(h1, wc0[...]) + bc0[...][None, :]


def _front_call(x_p, Wp1, bp1, Wp2, bp2, Wp3, bp3, parsing0, W1, b1, Wc0, bc0):
    blk = 640
    grid = N_PAD // blk
    full = lambda shape: pl.BlockSpec(shape, lambda i: (0,) * len(shape))
    return pl.pallas_call(
        _front_body,
        grid=(grid,),
        in_specs=[
            pl.BlockSpec((blk, D_IN), lambda i: (i, 0)),
            full((D_IN, 512)), full((512,)),
            full((512, 64)), full((64,)),
            full((64, D_OUT)), full((D_OUT,)),
            full((D_OUT, D_OUT)),
            full((D_IN, D_HID)), full((D_HID,)),
            full((D_HID, D_HID)), full((D_HID,)),
        ],
        out_specs=[
            pl.BlockSpec((blk, D_OUT), lambda i: (i, 0)),
            pl.BlockSpec((blk, D_OUT), lambda i: (i, 0)),
            pl.BlockSpec((blk, D_HID), lambda i: (i, 0)),
        ],
        out_shape=[
            jax.ShapeDtypeStruct((N_PAD, D_OUT), jnp.float32),
            jax.ShapeDtypeStruct((N_PAD, D_OUT), jnp.float32),
            jax.ShapeDtypeStruct((N_PAD, D_HID), jnp.float32),
        ],
    )(x_p, Wp1, bp1, Wp2, bp2, Wp3, bp3, parsing0, W1, b1, Wc0, bc0)


def _stats_body(ew_ref, ab_ref):
    ew = ew_ref[...]
    s1 = jnp.sum(ew)
    mean = s1 / E
    var = jnp.sum((ew - mean) ** 2) / (E - 1)
    a = jnp.sqrt(1e-4 / var)
    b = 1.0 - a * mean
    col = lax.broadcasted_iota(jnp.int32, (1, 128), 1)
    ab_ref[...] = jnp.where(col == 0, a, jnp.where(col == 1, b, 0.0))


def _stats_call(ew_valid):
    return pl.pallas_call(
        _stats_body,
        out_shape=jax.ShapeDtypeStruct((1, 128), jnp.float32),
    )(ew_valid)


def _finalize_body(degp_ref, cntp_ref, ab_ref, dis_ref, alpha_ref, beta_ref):
    a = ab_ref[0, 0]
    b = ab_ref[0, 1]
    deg = a * (degp_ref[0] + degp_ref[1]) + b * (cntp_ref[0] + cntp_ref[1])
    good = deg > 1e-6
    deg_c = jnp.maximum(deg, 1e-6)
    dis = jnp.where(good, lax.rsqrt(deg_c), 0.0)
    den = MU + jnp.where(good, 1.0, 0.0)
    alpha = 1.0 / den
    dis_ref[...] = dis
    alpha_ref[...] = alpha
    beta_ref[...] = MU * alpha


def _finalize_call(degp, cntp, ab):
    return pl.pallas_call(
        _finalize_body,
        out_shape=[
            jax.ShapeDtypeStruct((80, 128), jnp.float32),
            jax.ShapeDtypeStruct((80, 128), jnp.float32),
            jax.ShapeDtypeStruct((80, 128), jnp.float32),
        ],
    )(degp, cntp, ab)


def _combine_body(p0_ref, p1_ref, f0_ref, alpha_ref, beta_ref, out_ref):
    out_ref[...] = (alpha_ref[...] * (p0_ref[...] + p1_ref[...])
                    + beta_ref[...] * f0_ref[...])


def _combine_call(p0, p1, f0, alpha_c, beta_c, d):
    blk = 640
    return pl.pallas_call(
        _combine_body,
        grid=(N_PAD // blk,),
        in_specs=[
            pl.BlockSpec((blk, d), lambda i: (i, 0)),
            pl.BlockSpec((blk, d), lambda i: (i, 0)),
            pl.BlockSpec((blk, d), lambda i: (i, 0)),
            pl.BlockSpec((blk, 1), lambda i: (i, 0)),
            pl.BlockSpec((blk, 1), lambda i: (i, 0)),
        ],
        out_specs=pl.BlockSpec((blk, d), lambda i: (i, 0)),
        out_shape=jax.ShapeDtypeStruct((N_PAD, d), jnp.float32),
    )(p0, p1, f0, alpha_c, beta_c)


def _comb_mm_body(p0_ref, p1_ref, f0_ref, alpha_ref, beta_ref, wc1, bc1, out_ref):
    f2 = (alpha_ref[...] * (p0_ref[...] + p1_ref[...])
          + beta_ref[...] * f0_ref[...])
    h2 = jnp.maximum(f2, 0.0)
    out_ref[...] = _dot(h2, wc1[...]) + bc1[...][None, :]


def _comb_mm_call(p0, p1, f0, alpha_c, beta_c, Wc1, bc1):
    blk = 640
    return pl.pallas_call(
        _comb_mm_body,
        grid=(N_PAD // blk,),
        in_specs=[
            pl.BlockSpec((blk, D_HID), lambda i: (i, 0)),
            pl.BlockSpec((blk, D_HID), lambda i: (i, 0)),
            pl.BlockSpec((blk, D_HID), lambda i: (i, 0)),
            pl.BlockSpec((blk, 1), lambda i: (i, 0)),
            pl.BlockSpec((blk, 1), lambda i: (i, 0)),
            pl.BlockSpec((D_HID, D_OUT), lambda i: (0, 0)),
            pl.BlockSpec((D_OUT,), lambda i: (0,)),
        ],
        out_specs=pl.BlockSpec((blk, D_OUT), lambda i: (i, 0)),
        out_shape=jax.ShapeDtypeStruct((N_PAD, D_OUT), jnp.float32),
    )(p0, p1, f0, alpha_c, beta_c, Wc1, bc1)


_MESH = plsc.VectorSubcoreMesh(core_axis_name="c", subcore_axis_name="s",
                               num_cores=NC, num_subcores=NS)


def _iota16():
    return lax.iota(jnp.int32, LL)


def _zero_vec_ref(ref, n):
    z = jnp.zeros((LL,), jnp.float32)

    @pl.loop(0, n // LL)
    def _(i):
        ref[pl.ds(i * LL, LL)] = z


def _ew_deg_body(logits_hbm, q_hbm, row_hbm, col_hbm,
                 ew_hbm, degp_hbm, cntp_hbm,
                 ridx, cidx, abuf, bbuf, ewbuf, obuf, zbuf,
                 deg_sh, cnt_sh):
    cid = lax.axis_index("c")
    sid = lax.axis_index("s")
    wid = cid * NS + sid
    base = wid * NCH

    pltpu.sync_copy(row_hbm.at[pl.ds(base, NCH)], ridx)
    pltpu.sync_copy(col_hbm.at[pl.ds(base, NCH)], cidx)

    _zero_vec_ref(zbuf, NPT)
    pltpu.sync_copy(zbuf, deg_sh.at[pl.ds(sid * NPT, NPT)])
    pltpu.sync_copy(zbuf, cnt_sh.at[pl.ds(sid * NPT, NPT)])

    ones = jnp.ones((LL,), jnp.float32)

    @pl.loop(0, ECH // LL)
    def _(i):
        obuf[pl.ds(i * LL, LL)] = ones

    plsc.subcore_barrier()

    @pl.loop(0, NCH)
    def _(j):
        rj = ridx.at[j]
        cj = cidx.at[j]
        pltpu.sync_copy(logits_hbm.at[rj], abuf)
        pltpu.sync_copy(q_hbm.at[cj], bbuf)
        for g in range(ECH // LL):
            erow = g * LL + _iota16()
            acc = jnp.zeros((LL,), jnp.float32)
            for c in range(D_OUT):
                fcol = jnp.full((LL,), c, jnp.int32)
                av = plsc.load_gather(abuf, [erow, fcol])
                bv = plsc.load_gather(bbuf, [erow, fcol])
                acc = acc + av * bv
            ewbuf[pl.ds(g * LL, LL)] = acc
        pltpu.sync_copy(ewbuf, ew_hbm.at[base + j])
        pltpu.sync_copy(ewbuf, deg_sh.at[rj], add=True)
        pltpu.sync_copy(obuf, cnt_sh.at[rj], add=True)

    plsc.subcore_barrier()
    pltpu.sync_copy(deg_sh.at[pl.ds(sid * NPT, NPT)],
                    degp_hbm.at[cid, pl.ds(sid * NPT, NPT)])
    pltpu.sync_copy(cnt_sh.at[pl.ds(sid * NPT, NPT)],
                    cntp_hbm.at[cid, pl.ds(sid * NPT, NPT)])


def _ew_deg_call(logits_p, q_p, row2d, col2d):
    kern = pl.kernel(
        _ew_deg_body,
        out_type=[
            jax.ShapeDtypeStruct((E_PAD // ECH, ECH), jnp.float32),
            jax.ShapeDtypeStruct((NC, N_PAD), jnp.float32),
            jax.ShapeDtypeStruct((NC, N_PAD), jnp.float32),
        ],
        mesh=_MESH,
        scratch_types=[
            pltpu.VMEM((NCH, ECH), jnp.int32),
            pltpu.VMEM((NCH, ECH), jnp.int32),
            pltpu.VMEM((ECH, D_OUT), jnp.float32),
            pltpu.VMEM((ECH, D_OUT), jnp.float32),
            pltpu.VMEM((ECH,), jnp.float32),
            pltpu.VMEM((ECH,), jnp.float32),
            pltpu.VMEM((NPT,), jnp.float32),
            pltpu.VMEM_SHARED((N_PAD,), jnp.float32),
            pltpu.VMEM_SHARED((N_PAD,), jnp.float32),
        ],
    )
    return kern(logits_p, q_p, row2d, col2d)


def _coef_body(ew_hbm, row_hbm, col_hbm, dis_hbm, ab_hbm, coef_hbm,
               ridx, cidx, ewbuf, drbuf, dcbuf, cfbuf, abv):
    cid = lax.axis_index("c")
    sid = lax.axis_index("s")
    wid = cid * NS + sid
    base = wid * NCH

    pltpu.sync_copy(row_hbm.at[pl.ds(base, NCH)], ridx)
    pltpu.sync_copy(col_hbm.at[pl.ds(base, NCH)], cidx)
    pltpu.sync_copy(ab_hbm, abv)
    av = plsc.load_gather(abv, [jnp.zeros((LL,), jnp.int32)])
    bv = plsc.load_gather(abv, [jnp.ones((LL,), jnp.int32)])

    @pl.loop(0, NCH)
    def _(j):
        rj = ridx.at[j]
        cj = cidx.at[j]
        pltpu.sync_copy(ew_hbm.at[base + j], ewbuf)
        pltpu.sync_copy(dis_hbm.at[rj], drbuf)
        pltpu.sync_copy(dis_hbm.at[cj], dcbuf)
        for g in range(ECH // LL):
            sl = pl.ds(g * LL, LL)
            ewv = ewbuf[sl]
            cfbuf[sl] = (av * ewv + bv) * drbuf[sl] * dcbuf[sl]
        pltpu.sync_copy(cfbuf, coef_hbm.at[base + j])


def _coef_call(ew2d, row2d, col2d, dis_flat, ab16):
    kern = pl.kernel(
        _coef_body,
        out_type=jax.ShapeDtypeStruct((E_PAD // ECH, ECH), jnp.float32),
        mesh=_MESH,
        scratch_types=[
            pltpu.VMEM((NCH, ECH), jnp.int32),
            pltpu.VMEM((NCH, ECH), jnp.int32),
            pltpu.VMEM((ECH,), jnp.float32),
            pltpu.VMEM((ECH,), jnp.float32),
            pltpu.VMEM((ECH,), jnp.float32),
            pltpu.VMEM((ECH,), jnp.float32),
            pltpu.VMEM((LL,), jnp.float32),
        ],
    )
    return kern(ew2d, row2d, col2d, dis_flat, ab16)


def _spmv_body(d, f_hbm, row_hbm, col_hbm, coef_hbm, aggp_hbm,
               ridx, cidx, cvm, csm, rows, zbuf, agg_sh):
    cid = lax.axis_index("c")
    sid = lax.axis_index("s")
    wid = cid * NS + sid
    base = wid * NCH

    pltpu.sync_copy(row_hbm.at[pl.ds(base, NCH)], ridx)
    pltpu.sync_copy(col_hbm.at[pl.ds(base, NCH)], cidx)
    pltpu.sync_copy(coef_hbm.at[pl.ds(base, NCH)], cvm)

    z = jnp.zeros((LL,), jnp.float32)

    @pl.loop(0, 64)
    def _(i):
        for jj in range(d // LL):
            zbuf[i, pl.ds(jj * LL, LL)] = z

    @pl.loop(0, NPT // 64)
    def _(k):
        pltpu.sync_copy(zbuf, agg_sh.at[pl.ds(sid * NPT + k * 64, 64)])

    plsc.subcore_barrier()

    @pl.loop(0, NCH)
    def _(j):
        pltpu.sync_copy(cvm.at[j], csm)
        pltpu.sync_copy(f_hbm.at[cidx.at[j]], rows)

        @pl.loop(0, ECH)
        def _(e):
            c = csm[e]
            for jj in range(d // LL):
                sl = pl.ds(jj * LL, LL)
                rows[e, sl] = rows[e, sl] * c

        pltpu.sync_copy(rows, agg_sh.at[ridx.at[j]], add=True)

    plsc.subcore_barrier()
    pltpu.sync_copy(agg_sh.at[pl.ds(sid * NPT, NPT)],
                    aggp_hbm.at[cid, pl.ds(sid * NPT, NPT)])


def _spmv_call(f, row2d, col2d, coef2d, d):
    kern = pl.kernel(
        functools.partial(_spmv_body, d),
        out_type=jax.ShapeDtypeStruct((NC, N_PAD, d), jnp.float32),
        mesh=_MESH,
        scratch_types=[
            pltpu.VMEM((NCH, ECH), jnp.int32),
            pltpu.VMEM((NCH, ECH), jnp.int32),
            pltpu.VMEM((NCH, ECH), jnp.float32),
            pltpu.SMEM((ECH,), jnp.float32),
            pltpu.VMEM((ECH, d), jnp.float32),
            pltpu.VMEM((64, d), jnp.float32),
            pltpu.VMEM_SHARED((N_PAD, d), jnp.float32),
        ],
    )
    return kern(f, row2d, col2d, coef2d)


def kernel(x, edge_index, W1, b1, Wp1, bp1, Wp2, bp2, Wp3, bp3, parsing0,
           Wc0, bc0, Wc1, bc1):
    x_p = jnp.pad(x, ((0, N_PAD - N), (0, 0)))
    row = edge_index[0]
    col = edge_index[1]
    row_p = jnp.concatenate(
        [row, jnp.full((E_PAD - E,), N, jnp.int32)]).reshape(E_PAD // ECH, ECH)
    col_p = jnp.concatenate(
        [col, jnp.zeros((E_PAD - E,), jnp.int32)]).reshape(E_PAD // ECH, ECH)

    logits_p, q_p, f0c1 = _front_call(
        x_p, Wp1, bp1, Wp2, bp2, Wp3, bp3, parsing0, W1, b1, Wc0, bc0)

    ew2d, degp, cntp = _ew_deg_call(logits_p, q_p, row_p, col_p)

    ew_valid = ew2d.reshape(-1)[:E].reshape(E // 128, 128)
    ab = _stats_call(ew_valid)

    dis, alpha, beta = _finalize_call(
        degp.reshape(NC, N_PAD // 128, 128),
        cntp.reshape(NC, N_PAD // 128, 128), ab)

    dis_flat = dis.reshape(N_PAD)
    ab16 = ab.reshape(128)[:16]
    coef2d = _coef_call(ew2d, row_p, col_p, dis_flat, ab16)

    alpha_c = alpha.reshape(N_PAD, 1)
    beta_c = beta.reshape(N_PAD, 1)

    p = _spmv_call(f0c1, row_p, col_p, coef2d, D_HID)
    f1 = _combine_call(p[0], p[1], f0c1, alpha_c, beta_c, D_HID)
    p = _spmv_call(f1, row_p, col_p, coef2d, D_HID)
    g0 = _comb_mm_call(p[0], p[1], f0c1, alpha_c, beta_c, Wc1, bc1)

    p = _spmv_call(g0, row_p, col_p, coef2d, D_OUT)
    g1 = _combine_call(p[0], p[1], g0, alpha_c, beta_c, D_OUT)
    p = _spmv_call(g1, row_p, col_p, coef2d, D_OUT)
    g2 = _combine_call(p[0], p[1], g0, alpha_c, beta_c, D_OUT)

    return g2[:N]


# speedup vs baseline: 5.4305x; 5.4305x over previous
import dataclasses
import functools

import jax
import jax.numpy as jnp
from jax import lax
from jax.experimental import pallas as pl
from jax.experimental.pallas import tpu as pltpu
from jax.experimental.pallas import tpu_sc as plsc

N = 10000
E = 160000
D_IN, D_HID, D_OUT = 256, 128, 16
MU = 0.1
SCALING = 2.0

NC, NS, LL = 2, 16, 16
NW = NC * NS
N_PAD = 10240
E_PAD = 163840
EPT = E_PAD // NW
ECH = 128
NCH = EPT // ECH
NPT = N_PAD // NS

_P_HIGH = lax.Precision.HIGHEST


def _dot(a, b):
    return lax.dot_general(a, b, (((1,), (0,)), ((), ())),
                           precision=_P_HIGH, preferred_element_type=jnp.float32)


def _front_body(x_ref, wp1, bp1, wp2, bp2, wp3, bp3, pars, w1, b1, wc0, bc0,
                logits_ref, q_ref, f0_ref):
    xb = x_ref[...]
    hp = jnp.maximum(_dot(xb, wp1[...]) + bp1[...][None, :], 0.0)
    hp = jnp.maximum(_dot(hp, wp2[...]) + bp2[...][None, :], 0.0)
    logits = _dot(hp, wp3[...]) + bp3[...][None, :]
    logits_ref[...] = logits
    parsing = jnp.maximum(SCALING * pars[...], 0.0)
    q_ref[...] = _dot(logits, parsing)
    h1 = _dot(xb, w1[...]) + b1[...][None, :]
    f0_ref[...] = _dot(h1, wc0[...]) + bc0[...][None, :]


def _front_call(x_p, Wp1, bp1, Wp2, bp2, Wp3, bp3, parsing0, W1, b1, Wc0, bc0):
    blk = 640
    grid = N_PAD // blk
    full = lambda shape: pl.BlockSpec(shape, lambda i: (0,) * len(shape))
    return pl.pallas_call(
        _front_body,
        grid=(grid,),
        in_specs=[
            pl.BlockSpec((blk, D_IN), lambda i: (i, 0)),
            full((D_IN, 512)), full((512,)),
            full((512, 64)), full((64,)),
            full((64, D_OUT)), full((D_OUT,)),
            full((D_OUT, D_OUT)),
            full((D_IN, D_HID)), full((D_HID,)),
            full((D_HID, D_HID)), full((D_HID,)),
        ],
        out_specs=[
            pl.BlockSpec((blk, D_OUT), lambda i: (i, 0)),
            pl.BlockSpec((blk, D_OUT), lambda i: (i, 0)),
            pl.BlockSpec((blk, D_HID), lambda i: (i, 0)),
        ],
        out_shape=[
            jax.ShapeDtypeStruct((N_PAD, D_OUT), jnp.float32),
            jax.ShapeDtypeStruct((N_PAD, D_OUT), jnp.float32),
            jax.ShapeDtypeStruct((N_PAD, D_HID), jnp.float32),
        ],
    )(x_p, Wp1, bp1, Wp2, bp2, Wp3, bp3, parsing0, W1, b1, Wc0, bc0)


def _stats_body(ew_ref, ab_ref):
    ew = ew_ref[...]
    s1 = jnp.sum(ew)
    mean = s1 / E
    var = jnp.sum((ew - mean) ** 2) / (E - 1)
    a = jnp.sqrt(1e-4 / var)
    b = 1.0 - a * mean
    col = lax.broadcasted_iota(jnp.int32, (1, 128), 1)
    ab_ref[...] = jnp.where(col == 0, a, jnp.where(col == 1, b, 0.0))


def _stats_call(ew_valid):
    return pl.pallas_call(
        _stats_body,
        out_shape=jax.ShapeDtypeStruct((1, 128), jnp.float32),
    )(ew_valid)


def _finalize_body(degp_ref, cntp_ref, ab_ref, dis_ref, alpha_ref, beta_ref):
    a = ab_ref[0, 0]
    b = ab_ref[0, 1]
    deg = a * (degp_ref[0] + degp_ref[1]) + b * (cntp_ref[0] + cntp_ref[1])
    good = deg > 1e-6
    deg_c = jnp.maximum(deg, 1e-6)
    dis = jnp.where(good, lax.rsqrt(deg_c), 0.0)
    den = MU + jnp.where(good, 1.0, 0.0)
    alpha = 1.0 / den
    dis_ref[...] = dis
    alpha_ref[...] = alpha
    beta_ref[...] = MU * alpha


def _finalize_call(degp, cntp, ab):
    return pl.pallas_call(
        _finalize_body,
        out_shape=[
            jax.ShapeDtypeStruct((80, 128), jnp.float32),
            jax.ShapeDtypeStruct((80, 128), jnp.float32),
            jax.ShapeDtypeStruct((80, 128), jnp.float32),
        ],
    )(degp, cntp, ab)


def _combine_body(p0_ref, p1_ref, f0_ref, alpha_ref, beta_ref, out_ref):
    out_ref[...] = (alpha_ref[...] * (p0_ref[...] + p1_ref[...])
                    + beta_ref[...] * f0_ref[...])


def _combine_call(p0, p1, f0, alpha_c, beta_c, d):
    blk = 640
    return pl.pallas_call(
        _combine_body,
        grid=(N_PAD // blk,),
        in_specs=[
            pl.BlockSpec((blk, d), lambda i: (i, 0)),
            pl.BlockSpec((blk, d), lambda i: (i, 0)),
            pl.BlockSpec((blk, d), lambda i: (i, 0)),
            pl.BlockSpec((blk, 1), lambda i: (i, 0)),
            pl.BlockSpec((blk, 1), lambda i: (i, 0)),
        ],
        out_specs=pl.BlockSpec((blk, d), lambda i: (i, 0)),
        out_shape=jax.ShapeDtypeStruct((N_PAD, d), jnp.float32),
    )(p0, p1, f0, alpha_c, beta_c)


def _comb_mm_body(p0_ref, p1_ref, f0_ref, alpha_ref, beta_ref, wc1, bc1, out_ref):
    f2 = (alpha_ref[...] * (p0_ref[...] + p1_ref[...])
          + beta_ref[...] * f0_ref[...])
    h2 = jnp.maximum(f2, 0.0)
    out_ref[...] = _dot(h2, wc1[...]) + bc1[...][None, :]


def _comb_mm_call(p0, p1, f0, alpha_c, beta_c, Wc1, bc1):
    blk = 640
    return pl.pallas_call(
        _comb_mm_body,
        grid=(N_PAD // blk,),
        in_specs=[
            pl.BlockSpec((blk, D_HID), lambda i: (i, 0)),
            pl.BlockSpec((blk, D_HID), lambda i: (i, 0)),
            pl.BlockSpec((blk, D_HID), lambda i: (i, 0)),
            pl.BlockSpec((blk, 1), lambda i: (i, 0)),
            pl.BlockSpec((blk, 1), lambda i: (i, 0)),
            pl.BlockSpec((D_HID, D_OUT), lambda i: (0, 0)),
            pl.BlockSpec((D_OUT,), lambda i: (0,)),
        ],
        out_specs=pl.BlockSpec((blk, D_OUT), lambda i: (i, 0)),
        out_shape=jax.ShapeDtypeStruct((N_PAD, D_OUT), jnp.float32),
    )(p0, p1, f0, alpha_c, beta_c, Wc1, bc1)


_MESH = plsc.VectorSubcoreMesh(core_axis_name="c", subcore_axis_name="s",
                               num_cores=NC, num_subcores=NS)

_SC_PARAMS = pltpu.CompilerParams()
if "needs_layout_passes" in pltpu.CompilerParams.__dataclass_fields__:
    _SC_PARAMS = dataclasses.replace(_SC_PARAMS, needs_layout_passes=False)
if "use_tc_tiling_on_sc" in pltpu.CompilerParams.__dataclass_fields__:
    _SC_PARAMS = dataclasses.replace(_SC_PARAMS, use_tc_tiling_on_sc=False)


def _iota16():
    return lax.iota(jnp.int32, LL)


def _zero_vec_ref(ref, n):
    z = jnp.zeros((LL,), jnp.float32)

    @pl.loop(0, n // LL)
    def _(i):
        ref[pl.ds(i * LL, LL)] = z


def _ew_deg_body(logits_hbm, q_hbm, row_hbm, col_hbm,
                 ew_hbm, degp_hbm, cntp_hbm,
                 ridx, cidx, abuf, bbuf, ewbuf, obuf, zbuf,
                 deg_sh, cnt_sh):
    cid = lax.axis_index("c")
    sid = lax.axis_index("s")
    wid = cid * NS + sid
    base = wid * NCH

    pltpu.sync_copy(row_hbm.at[pl.ds(base, NCH)], ridx)
    pltpu.sync_copy(col_hbm.at[pl.ds(base, NCH)], cidx)

    _zero_vec_ref(zbuf, NPT)
    pltpu.sync_copy(zbuf, deg_sh.at[pl.ds(sid * NPT, NPT)])
    pltpu.sync_copy(zbuf, cnt_sh.at[pl.ds(sid * NPT, NPT)])

    ones = jnp.ones((LL,), jnp.float32)

    @pl.loop(0, ECH // LL)
    def _(i):
        obuf[pl.ds(i * LL, LL)] = ones

    plsc.subcore_barrier()

    @pl.loop(0, NCH)
    def _(j):
        rj = ridx.at[j]
        cj = cidx.at[j]
        pltpu.sync_copy(logits_hbm.at[rj], abuf)
        pltpu.sync_copy(q_hbm.at[cj], bbuf)
        for g in range(ECH // LL):
            erow = g * LL + _iota16()
            acc = jnp.zeros((LL,), jnp.float32)
            for c in range(D_OUT):
                fcol = jnp.full((LL,), c, jnp.int32)
                av = plsc.load_gather(abuf, [erow, fcol])
                bv = plsc.load_gather(bbuf, [erow, fcol])
                acc = acc + av * bv
            ewbuf[pl.ds(g * LL, LL)] = acc
        pltpu.sync_copy(ewbuf, ew_hbm.at[base + j])
        pltpu.sync_copy(ewbuf, deg_sh.at[rj], add=True)
        pltpu.sync_copy(obuf, cnt_sh.at[rj], add=True)

    plsc.subcore_barrier()
    pltpu.sync_copy(deg_sh.at[pl.ds(sid * NPT, NPT)],
                    degp_hbm.at[cid, pl.ds(sid * NPT, NPT)])
    pltpu.sync_copy(cnt_sh.at[pl.ds(sid * NPT, NPT)],
                    cntp_hbm.at[cid, pl.ds(sid * NPT, NPT)])


def _ew_deg_call(logits_p, q_p, row2d, col2d):
    kern = pl.kernel(
        _ew_deg_body,
        out_type=[
            jax.ShapeDtypeStruct((E_PAD // ECH, ECH), jnp.float32),
            jax.ShapeDtypeStruct((NC, N_PAD), jnp.float32),
            jax.ShapeDtypeStruct((NC, N_PAD), jnp.float32),
        ],
        mesh=_MESH,
        compiler_params=_SC_PARAMS,
        scratch_types=[
            pltpu.VMEM((NCH, ECH), jnp.int32),
            pltpu.VMEM((NCH, ECH), jnp.int32),
            pltpu.VMEM((ECH, D_OUT), jnp.float32),
            pltpu.VMEM((ECH, D_OUT), jnp.float32),
            pltpu.VMEM((ECH,), jnp.float32),
            pltpu.VMEM((ECH,), jnp.float32),
            pltpu.VMEM((NPT,), jnp.float32),
            pltpu.VMEM_SHARED((N_PAD,), jnp.float32),
            pltpu.VMEM_SHARED((N_PAD,), jnp.float32),
        ],
    )
    return kern(logits_p, q_p, row2d, col2d)


def _coef_body(ew_hbm, row_hbm, col_hbm, dis_hbm, a_hbm, b_hbm, coef_hbm,
               ridx, cidx, ewbuf, cfbuf, disv, avr, bvr):
    cid = lax.axis_index("c")
    sid = lax.axis_index("s")
    wid = cid * NS + sid
    base = wid * NCH

    pltpu.sync_copy(row_hbm.at[pl.ds(base, NCH)], ridx)
    pltpu.sync_copy(col_hbm.at[pl.ds(base, NCH)], cidx)
    pltpu.sync_copy(dis_hbm, disv)
    pltpu.sync_copy(a_hbm, avr)
    pltpu.sync_copy(b_hbm, bvr)
    av = avr[...]
    bv = bvr[...]

    @pl.loop(0, NCH)
    def _(j):
        pltpu.sync_copy(ew_hbm.at[base + j], ewbuf)
        for g in range(ECH // LL):
            sl = pl.ds(g * LL, LL)
            rv = ridx[j, sl]
            cv = cidx[j, sl]
            dr = plsc.load_gather(disv, [rv])
            dc = plsc.load_gather(disv, [cv])
            cfbuf[sl] = (av * ewbuf[sl] + bv) * dr * dc
        pltpu.sync_copy(cfbuf, coef_hbm.at[base + j])


def _coef_call(ew2d, row2d, col2d, dis_flat, a16, b16):
    kern = pl.kernel(
        _coef_body,
        out_type=jax.ShapeDtypeStruct((E_PAD // ECH, ECH), jnp.float32),
        mesh=_MESH,
        compiler_params=_SC_PARAMS,
        scratch_types=[
            pltpu.VMEM((NCH, ECH), jnp.int32),
            pltpu.VMEM((NCH, ECH), jnp.int32),
            pltpu.VMEM((ECH,), jnp.float32),
            pltpu.VMEM((ECH,), jnp.float32),
            pltpu.VMEM((N_PAD,), jnp.float32),
            pltpu.VMEM((LL,), jnp.float32),
            pltpu.VMEM((LL,), jnp.float32),
        ],
    )
    return kern(ew2d, row2d, col2d, dis_flat, a16, b16)


def _spmv_body(d, f_hbm, row_hbm, col_hbm, coef_hbm, aggp_hbm,
               ridx, cidx, cvm, rows, zbuf, agg_sh):
    cid = lax.axis_index("c")
    sid = lax.axis_index("s")
    wid = cid * NS + sid
    base = wid * NCH

    pltpu.sync_copy(row_hbm.at[pl.ds(base, NCH)], ridx)
    pltpu.sync_copy(col_hbm.at[pl.ds(base, NCH)], cidx)
    pltpu.sync_copy(coef_hbm.at[pl.ds(base, NCH)], cvm)

    z = jnp.zeros((LL,), jnp.float32)

    @pl.loop(0, 64)
    def _(i):
        for jj in range(d // LL):
            zbuf[i, pl.ds(jj * LL, LL)] = z

    @pl.loop(0, NPT // 64)
    def _(k):
        pltpu.sync_copy(zbuf, agg_sh.at[pl.ds(sid * NPT + k * 64, 64)])

    plsc.subcore_barrier()

    @pl.loop(0, NCH)
    def _(j):
        pltpu.sync_copy(f_hbm.at[cidx.at[j]], rows)
        jv = jnp.full((LL,), j, jnp.int32)

        @pl.loop(0, ECH)
        def _(e):
            cb = plsc.load_gather(cvm, [jv, jnp.full((LL,), e, jnp.int32)])
            for jj in range(d // LL):
                sl = pl.ds(jj * LL, LL)
                rows[e, sl] = rows[e, sl] * cb

        pltpu.sync_copy(rows, agg_sh.at[ridx.at[j]], add=True)

    plsc.subcore_barrier()
    pltpu.sync_copy(agg_sh.at[pl.ds(sid * NPT, NPT)],
                    aggp_hbm.at[cid, pl.ds(sid * NPT, NPT)])


def _spmv_call(f, row2d, col2d, coef2d, d):
    kern = pl.kernel(
        functools.partial(_spmv_body, d),
        out_type=jax.ShapeDtypeStruct((NC, N_PAD, d), jnp.float32),
        mesh=_MESH,
        compiler_params=_SC_PARAMS,
        scratch_types=[
            pltpu.VMEM((NCH, ECH), jnp.int32),
            pltpu.VMEM((NCH, ECH), jnp.int32),
            pltpu.VMEM((NCH, ECH), jnp.float32),
            pltpu.VMEM((ECH, d), jnp.float32),
            pltpu.VMEM((64, d), jnp.float32),
            pltpu.VMEM_SHARED((N_PAD, d), jnp.float32),
        ],
    )
    return kern(f, row2d, col2d, coef2d)


def kernel(x, edge_index, W1, b1, Wp1, bp1, Wp2, bp2, Wp3, bp3, parsing0,
           Wc0, bc0, Wc1, bc1):
    x_p = jnp.pad(x, ((0, N_PAD - N), (0, 0)))
    row = edge_index[0]
    col = edge_index[1]
    row_p = jnp.concatenate(
        [row, jnp.full((E_PAD - E,), N, jnp.int32)]).reshape(E_PAD // ECH, ECH)
    col_p = jnp.concatenate(
        [col, jnp.zeros((E_PAD - E,), jnp.int32)]).reshape(E_PAD // ECH, ECH)

    logits_p, q_p, f0c1 = _front_call(
        x_p, Wp1, bp1, Wp2, bp2, Wp3, bp3, parsing0, W1, b1, Wc0, bc0)

    ew2d, degp, cntp = _ew_deg_call(logits_p, q_p, row_p, col_p)

    ew_valid = ew2d.reshape(-1)[:E].reshape(E // 128, 128)
    ab = _stats_call(ew_valid)

    dis, alpha, beta = _finalize_call(
        degp.reshape(NC, N_PAD // 128, 128),
        cntp.reshape(NC, N_PAD // 128, 128), ab)

    dis_flat = dis.reshape(N_PAD)
    a16 = jnp.broadcast_to(ab.reshape(128)[0], (LL,))
    b16 = jnp.broadcast_to(ab.reshape(128)[1], (LL,))
    coef2d = _coef_call(ew2d, row_p, col_p, dis_flat, a16, b16)

    alpha_c = alpha.reshape(N_PAD, 1)
    beta_c = beta.reshape(N_PAD, 1)

    def spmv(f, d):
        p = _spmv_call(f, row_p, col_p, coef2d, d)
        return p[0], p[1]

    p0, p1 = spmv(f0c1, D_HID)
    f1 = _combine_call(p0, p1, f0c1, alpha_c, beta_c, D_HID)
    p0, p1 = spmv(f1, D_HID)
    g0 = _comb_mm_call(p0, p1, f0c1, alpha_c, beta_c, Wc1, bc1)

    p0, p1 = spmv(g0, D_OUT)
    g1 = _combine_call(p0, p1, g0, alpha_c, beta_c, D_OUT)
    p0, p1 = spmv(g1, D_OUT)
    g2 = _combine_call(p0, p1, g0, alpha_c, beta_c, D_OUT)

    return g2[:N]


# speedup vs baseline: 6.7254x; 1.2384x over previous
import dataclasses
import functools

import jax
import jax.numpy as jnp
from jax import lax
from jax.experimental import pallas as pl
from jax.experimental.pallas import tpu as pltpu
from jax.experimental.pallas import tpu_sc as plsc

N = 10000
E = 160000
D_IN, D_HID, D_OUT = 256, 128, 16
MU = 0.1
SCALING = 2.0

NC, NS, LL = 2, 16, 16
NW = NC * NS
N_PAD = 10240
E_PAD = 163840
EPT = E_PAD // NW
ECH = 128
NCH = EPT // ECH
NPT = N_PAD // NS

_P_HIGH = lax.Precision.HIGHEST


def _dot(a, b):
    return lax.dot_general(a, b, (((1,), (0,)), ((), ())),
                           precision=_P_HIGH, preferred_element_type=jnp.float32)


def _front_body(x_ref, wp1, bp1, wp2, bp2, wp3, bp3, pars, w1, b1, wc0, bc0,
                logits_ref, q_ref, f0_ref):
    xb = x_ref[...]
    hp = jnp.maximum(_dot(xb, wp1[...]) + bp1[...][None, :], 0.0)
    hp = jnp.maximum(_dot(hp, wp2[...]) + bp2[...][None, :], 0.0)
    logits = _dot(hp, wp3[...]) + bp3[...][None, :]
    logits_ref[...] = logits
    parsing = jnp.maximum(SCALING * pars[...], 0.0)
    q_ref[...] = _dot(logits, parsing)
    h1 = _dot(xb, w1[...]) + b1[...][None, :]
    f0_ref[...] = _dot(h1, wc0[...]) + bc0[...][None, :]


def _front_call(x_p, Wp1, bp1, Wp2, bp2, Wp3, bp3, parsing0, W1, b1, Wc0, bc0):
    blk = 640
    grid = N_PAD // blk
    full = lambda shape: pl.BlockSpec(shape, lambda i: (0,) * len(shape))
    return pl.pallas_call(
        _front_body,
        grid=(grid,),
        in_specs=[
            pl.BlockSpec((blk, D_IN), lambda i: (i, 0)),
            full((D_IN, 512)), full((512,)),
            full((512, 64)), full((64,)),
            full((64, D_OUT)), full((D_OUT,)),
            full((D_OUT, D_OUT)),
            full((D_IN, D_HID)), full((D_HID,)),
            full((D_HID, D_HID)), full((D_HID,)),
        ],
        out_specs=[
            pl.BlockSpec((blk, D_OUT), lambda i: (i, 0)),
            pl.BlockSpec((blk, D_OUT), lambda i: (i, 0)),
            pl.BlockSpec((blk, D_HID), lambda i: (i, 0)),
        ],
        out_shape=[
            jax.ShapeDtypeStruct((N_PAD, D_OUT), jnp.float32),
            jax.ShapeDtypeStruct((N_PAD, D_OUT), jnp.float32),
            jax.ShapeDtypeStruct((N_PAD, D_HID), jnp.float32),
        ],
    )(x_p, Wp1, bp1, Wp2, bp2, Wp3, bp3, parsing0, W1, b1, Wc0, bc0)


def _stats_body(ew_ref, ab_ref):
    ew = ew_ref[...]
    s1 = jnp.sum(ew)
    mean = s1 / E
    var = jnp.sum((ew - mean) ** 2) / (E - 1)
    a = jnp.sqrt(1e-4 / var)
    b = 1.0 - a * mean
    col = lax.broadcasted_iota(jnp.int32, (1, 128), 1)
    ab_ref[...] = jnp.where(col == 0, a, jnp.where(col == 1, b, 0.0))


def _stats_call(ew_valid):
    return pl.pallas_call(
        _stats_body,
        out_shape=jax.ShapeDtypeStruct((1, 128), jnp.float32),
    )(ew_valid)


def _finalize_body(degp_ref, cntp_ref, ab_ref, dis_ref, alpha_ref, beta_ref):
    a = ab_ref[0, 0]
    b = ab_ref[0, 1]
    deg = a * (degp_ref[0] + degp_ref[1]) + b * (cntp_ref[0] + cntp_ref[1])
    good = deg > 1e-6
    deg_c = jnp.maximum(deg, 1e-6)
    dis = jnp.where(good, lax.rsqrt(deg_c), 0.0)
    den = MU + jnp.where(good, 1.0, 0.0)
    alpha = 1.0 / den
    dis_ref[...] = dis
    alpha_ref[...] = alpha
    beta_ref[...] = MU * alpha


def _finalize_call(degp, cntp, ab):
    return pl.pallas_call(
        _finalize_body,
        out_shape=[
            jax.ShapeDtypeStruct((80, 128), jnp.float32),
            jax.ShapeDtypeStruct((80, 128), jnp.float32),
            jax.ShapeDtypeStruct((80, 128), jnp.float32),
        ],
    )(degp, cntp, ab)


def _combine_body(p0_ref, p1_ref, f0_ref, alpha_ref, beta_ref, out_ref):
    out_ref[...] = (alpha_ref[...] * (p0_ref[...] + p1_ref[...])
                    + beta_ref[...] * f0_ref[...])


def _combine_call(p0, p1, f0, alpha_c, beta_c, d):
    blk = 640
    return pl.pallas_call(
        _combine_body,
        grid=(N_PAD // blk,),
        in_specs=[
            pl.BlockSpec((blk, d), lambda i: (i, 0)),
            pl.BlockSpec((blk, d), lambda i: (i, 0)),
            pl.BlockSpec((blk, d), lambda i: (i, 0)),
            pl.BlockSpec((blk, 1), lambda i: (i, 0)),
            pl.BlockSpec((blk, 1), lambda i: (i, 0)),
        ],
        out_specs=pl.BlockSpec((blk, d), lambda i: (i, 0)),
        out_shape=jax.ShapeDtypeStruct((N_PAD, d), jnp.float32),
    )(p0, p1, f0, alpha_c, beta_c)


def _comb_mm_body(p0_ref, p1_ref, f0_ref, alpha_ref, beta_ref, wc1, bc1, out_ref):
    f2 = (alpha_ref[...] * (p0_ref[...] + p1_ref[...])
          + beta_ref[...] * f0_ref[...])
    h2 = jnp.maximum(f2, 0.0)
    out_ref[...] = _dot(h2, wc1[...]) + bc1[...][None, :]


def _comb_mm_call(p0, p1, f0, alpha_c, beta_c, Wc1, bc1):
    blk = 640
    return pl.pallas_call(
        _comb_mm_body,
        grid=(N_PAD // blk,),
        in_specs=[
            pl.BlockSpec((blk, D_HID), lambda i: (i, 0)),
            pl.BlockSpec((blk, D_HID), lambda i: (i, 0)),
            pl.BlockSpec((blk, D_HID), lambda i: (i, 0)),
            pl.BlockSpec((blk, 1), lambda i: (i, 0)),
            pl.BlockSpec((blk, 1), lambda i: (i, 0)),
            pl.BlockSpec((D_HID, D_OUT), lambda i: (0, 0)),
            pl.BlockSpec((D_OUT,), lambda i: (0,)),
        ],
        out_specs=pl.BlockSpec((blk, D_OUT), lambda i: (i, 0)),
        out_shape=jax.ShapeDtypeStruct((N_PAD, D_OUT), jnp.float32),
    )(p0, p1, f0, alpha_c, beta_c, Wc1, bc1)


_MESH = plsc.VectorSubcoreMesh(core_axis_name="c", subcore_axis_name="s",
                               num_cores=NC, num_subcores=NS)

_SC_PARAMS = pltpu.CompilerParams()
if "needs_layout_passes" in pltpu.CompilerParams.__dataclass_fields__:
    _SC_PARAMS = dataclasses.replace(_SC_PARAMS, needs_layout_passes=False)
if "use_tc_tiling_on_sc" in pltpu.CompilerParams.__dataclass_fields__:
    _SC_PARAMS = dataclasses.replace(_SC_PARAMS, use_tc_tiling_on_sc=False)


def _iota16():
    return lax.iota(jnp.int32, LL)


def _zero_vec_ref(ref, n):
    z = jnp.zeros((LL,), jnp.float32)

    @pl.loop(0, n // LL)
    def _(i):
        ref[pl.ds(i * LL, LL)] = z


def _ew_deg_body(logits_hbm, q_hbm, row_hbm, col_hbm,
                 ew_hbm, degp_hbm, cntp_hbm,
                 ridx, cidx, abuf, bbuf, ewbuf, obuf, zbuf,
                 deg_sh, cnt_sh):
    cid = lax.axis_index("c")
    sid = lax.axis_index("s")
    wid = cid * NS + sid
    base = wid * NCH

    pltpu.sync_copy(row_hbm.at[pl.ds(base, NCH)], ridx)
    pltpu.sync_copy(col_hbm.at[pl.ds(base, NCH)], cidx)

    _zero_vec_ref(zbuf, NPT)
    pltpu.sync_copy(zbuf, deg_sh.at[pl.ds(sid * NPT, NPT)])
    pltpu.sync_copy(zbuf, cnt_sh.at[pl.ds(sid * NPT, NPT)])

    ones = jnp.ones((LL,), jnp.float32)

    @pl.loop(0, ECH // LL)
    def _(i):
        obuf[pl.ds(i * LL, LL)] = ones

    plsc.subcore_barrier()

    @pl.loop(0, NCH)
    def _(j):
        rj = ridx.at[j]
        cj = cidx.at[j]
        pltpu.sync_copy(logits_hbm.at[rj], abuf)
        pltpu.sync_copy(q_hbm.at[cj], bbuf)
        for g in range(ECH // LL):
            erow = g * LL + _iota16()
            acc = jnp.zeros((LL,), jnp.float32)
            for c in range(D_OUT):
                fcol = jnp.full((LL,), c, jnp.int32)
                av = plsc.load_gather(abuf, [erow, fcol])
                bv = plsc.load_gather(bbuf, [erow, fcol])
                acc = acc + av * bv
            ewbuf[pl.ds(g * LL, LL)] = acc
        pltpu.sync_copy(ewbuf, ew_hbm.at[base + j])
        pltpu.sync_copy(ewbuf, deg_sh.at[rj], add=True)
        pltpu.sync_copy(obuf, cnt_sh.at[rj], add=True)

    plsc.subcore_barrier()
    pltpu.sync_copy(deg_sh.at[pl.ds(sid * NPT, NPT)],
                    degp_hbm.at[cid, pl.ds(sid * NPT, NPT)])
    pltpu.sync_copy(cnt_sh.at[pl.ds(sid * NPT, NPT)],
                    cntp_hbm.at[cid, pl.ds(sid * NPT, NPT)])


def _ew_deg_call(logits_p, q_p, row2d, col2d):
    kern = pl.kernel(
        _ew_deg_body,
        out_type=[
            jax.ShapeDtypeStruct((E_PAD // ECH, ECH), jnp.float32),
            jax.ShapeDtypeStruct((NC, N_PAD), jnp.float32),
            jax.ShapeDtypeStruct((NC, N_PAD), jnp.float32),
        ],
        mesh=_MESH,
        compiler_params=_SC_PARAMS,
        scratch_types=[
            pltpu.VMEM((NCH, ECH), jnp.int32),
            pltpu.VMEM((NCH, ECH), jnp.int32),
            pltpu.VMEM((ECH, D_OUT), jnp.float32),
            pltpu.VMEM((ECH, D_OUT), jnp.float32),
            pltpu.VMEM((ECH,), jnp.float32),
            pltpu.VMEM((ECH,), jnp.float32),
            pltpu.VMEM((NPT,), jnp.float32),
            pltpu.VMEM_SHARED((N_PAD,), jnp.float32),
            pltpu.VMEM_SHARED((N_PAD,), jnp.float32),
        ],
    )
    return kern(logits_p, q_p, row2d, col2d)


def _coef_body(ew_hbm, row_hbm, col_hbm, dis_hbm, a_hbm, b_hbm, coef_hbm,
               ridx, cidx, ewbuf, cfbuf, disv, avr, bvr):
    cid = lax.axis_index("c")
    sid = lax.axis_index("s")
    wid = cid * NS + sid
    base = wid * NCH

    pltpu.sync_copy(row_hbm.at[pl.ds(base, NCH)], ridx)
    pltpu.sync_copy(col_hbm.at[pl.ds(base, NCH)], cidx)
    pltpu.sync_copy(dis_hbm, disv)
    pltpu.sync_copy(a_hbm, avr)
    pltpu.sync_copy(b_hbm, bvr)
    av = avr[...]
    bv = bvr[...]

    @pl.loop(0, NCH)
    def _(j):
        pltpu.sync_copy(ew_hbm.at[base + j], ewbuf)
        for g in range(ECH // LL):
            sl = pl.ds(g * LL, LL)
            rv = ridx[j, sl]
            cv = cidx[j, sl]
            dr = plsc.load_gather(disv, [rv])
            dc = plsc.load_gather(disv, [cv])
            cfbuf[sl] = (av * ewbuf[sl] + bv) * dr * dc
        pltpu.sync_copy(cfbuf, coef_hbm.at[base + j])


def _coef_call(ew2d, row2d, col2d, dis_flat, a16, b16):
    kern = pl.kernel(
        _coef_body,
        out_type=jax.ShapeDtypeStruct((E_PAD // ECH, ECH), jnp.float32),
        mesh=_MESH,
        compiler_params=_SC_PARAMS,
        scratch_types=[
            pltpu.VMEM((NCH, ECH), jnp.int32),
            pltpu.VMEM((NCH, ECH), jnp.int32),
            pltpu.VMEM((ECH,), jnp.float32),
            pltpu.VMEM((ECH,), jnp.float32),
            pltpu.VMEM((N_PAD,), jnp.float32),
            pltpu.VMEM((LL,), jnp.float32),
            pltpu.VMEM((LL,), jnp.float32),
        ],
    )
    return kern(ew2d, row2d, col2d, dis_flat, a16, b16)


def _spmv_body(d, nb, f_hbm, row_hbm, col_hbm, coef_hbm, aggp_hbm,
               ridx, cidx, cvm, rows, agg_sh, *sems):
    gsems = sems[:nb]
    ssems = sems[nb:]
    pd = nb // 2
    cid = lax.axis_index("c")
    sid = lax.axis_index("s")
    wid = cid * NS + sid
    base = wid * NCH

    pltpu.sync_copy(row_hbm.at[pl.ds(base, NCH)], ridx)
    pltpu.sync_copy(col_hbm.at[pl.ds(base, NCH)], cidx)
    pltpu.sync_copy(coef_hbm.at[pl.ds(base, NCH)], cvm)

    z = jnp.zeros((LL,), jnp.float32)

    @pl.loop(0, 64)
    def _(i):
        for jj in range(d // LL):
            rows[i, pl.ds(jj * LL, LL)] = z

    zsrc = rows.at[pl.ds(0, 64)]

    @pl.loop(0, NPT // 64)
    def _(k):
        pltpu.sync_copy(zsrc, agg_sh.at[pl.ds(sid * NPT + k * 64, 64)])

    plsc.subcore_barrier()

    def rows_at(b):
        return rows.at[pl.ds(b * ECH, ECH)]

    def issue_gather(k, b):
        pltpu.async_copy(f_hbm.at[cidx.at[k]], rows_at(b), gsems[b])

    def wait_gather(k, b):
        pltpu.make_async_copy(f_hbm.at[cidx.at[k]], rows_at(b), gsems[b]).wait()

    def issue_scatter(k, b):
        pltpu.async_copy(rows_at(b), agg_sh.at[ridx.at[k]], ssems[b], add=True)

    def wait_scatter(k, b):
        pltpu.make_async_copy(rows_at(b), agg_sh.at[ridx.at[k]],
                              ssems[b]).wait()

    def scale(k, b):
        kv = jnp.full((LL,), k, jnp.int32)

        @pl.loop(0, ECH, step=2)
        def _(e):
            e1 = e + 1
            cb0 = plsc.load_gather(cvm, [kv, jnp.full((LL,), e, jnp.int32)])
            cb1 = plsc.load_gather(cvm, [kv, jnp.full((LL,), e1, jnp.int32)])
            r = rows_at(b)
            for jj in range(d // LL):
                sl = pl.ds(jj * LL, LL)
                r[e, sl] = r[e, sl] * cb0
            for jj in range(d // LL):
                sl = pl.ds(jj * LL, LL)
                r[e1, sl] = r[e1, sl] * cb1

    for k in range(pd):
        issue_gather(k, k)

    @pl.loop(0, NCH // nb)
    def _(outer):
        k0 = outer * nb
        for i in range(nb):
            k = k0 + i
            wait_gather(k, i)
            bp = (i + pd) % nb
            kp = k + pd
            kold = kp - nb

            @pl.when(kp < NCH)
            def _():
                @pl.when(kold >= 0)
                def _():
                    wait_scatter(kold, bp)
                issue_gather(kp, bp)

            scale(k, i)
            issue_scatter(k, i)

    for k in range(NCH - nb, NCH):
        wait_scatter(k, k % nb)

    plsc.subcore_barrier()
    pltpu.sync_copy(agg_sh.at[pl.ds(sid * NPT, NPT)],
                    aggp_hbm.at[cid, pl.ds(sid * NPT, NPT)])


@functools.cache
def _spmv_kernel(d):
    nb = 2 if d == D_HID else 4
    sems = [pltpu.SemaphoreType.DMA] * (2 * nb)
    return pl.kernel(
        functools.partial(_spmv_body, d, nb),
        out_type=jax.ShapeDtypeStruct((NC, N_PAD, d), jnp.float32),
        mesh=_MESH,
        compiler_params=_SC_PARAMS,
        scratch_types=[
            pltpu.VMEM((NCH, ECH), jnp.int32),
            pltpu.VMEM((NCH, ECH), jnp.int32),
            pltpu.VMEM((NCH, ECH), jnp.float32),
            pltpu.VMEM((nb * ECH, d), jnp.float32),
            pltpu.VMEM_SHARED((N_PAD, d), jnp.float32),
        ] + sems,
    )


def _spmv_call(f, row2d, col2d, coef2d, d):
    return _spmv_kernel(d)(f, row2d, col2d, coef2d)


def kernel(x, edge_index, W1, b1, Wp1, bp1, Wp2, bp2, Wp3, bp3, parsing0,
           Wc0, bc0, Wc1, bc1):
    x_p = jnp.pad(x, ((0, N_PAD - N), (0, 0)))
    row = edge_index[0]
    col = edge_index[1]
    row_p = jnp.concatenate(
        [row, jnp.full((E_PAD - E,), N, jnp.int32)]).reshape(E_PAD // ECH, ECH)
    col_p = jnp.concatenate(
        [col, jnp.zeros((E_PAD - E,), jnp.int32)]).reshape(E_PAD // ECH, ECH)

    logits_p, q_p, f0c1 = _front_call(
        x_p, Wp1, bp1, Wp2, bp2, Wp3, bp3, parsing0, W1, b1, Wc0, bc0)

    ew2d, degp, cntp = _ew_deg_call(logits_p, q_p, row_p, col_p)

    ew_valid = ew2d.reshape(-1)[:E].reshape(E // 128, 128)
    ab = _stats_call(ew_valid)

    dis, alpha, beta = _finalize_call(
        degp.reshape(NC, N_PAD // 128, 128),
        cntp.reshape(NC, N_PAD // 128, 128), ab)

    dis_flat = dis.reshape(N_PAD)
    a16 = jnp.broadcast_to(ab.reshape(128)[0], (LL,))
    b16 = jnp.broadcast_to(ab.reshape(128)[1], (LL,))
    coef2d = _coef_call(ew2d, row_p, col_p, dis_flat, a16, b16)

    alpha_c = alpha.reshape(N_PAD, 1)
    beta_c = beta.reshape(N_PAD, 1)

    def spmv(f, d):
        p = _spmv_call(f, row_p, col_p, coef2d, d)
        return p[0], p[1]

    p0, p1 = spmv(f0c1, D_HID)
    f1 = _combine_call(p0, p1, f0c1, alpha_c, beta_c, D_HID)
    p0, p1 = spmv(f1, D_HID)
    g0 = _comb_mm_call(p0, p1, f0c1, alpha_c, beta_c, Wc1, bc1)

    p0, p1 = spmv(g0, D_OUT)
    g1 = _combine_call(p0, p1, g0, alpha_c, beta_c, D_OUT)
    p0, p1 = spmv(g1, D_OUT)
    g2 = _combine_call(p0, p1, g0, alpha_c, beta_c, D_OUT)

    return g2[:N]


# speedup vs baseline: 7.4095x; 1.1017x over previous
import dataclasses
import functools

import jax
import jax.numpy as jnp
from jax import lax
from jax.experimental import pallas as pl
from jax.experimental.pallas import tpu as pltpu
from jax.experimental.pallas import tpu_sc as plsc

N = 10000
E = 160000
D_IN, D_HID, D_OUT = 256, 128, 16
MU = 0.1
SCALING = 2.0

NC, NS, LL = 2, 16, 16
NW = NC * NS
N_PAD = 10240
E_PAD = 163840
EPT = E_PAD // NW
ECH = 128
NCH = EPT // ECH
NPT = N_PAD // NS

_P_HIGH = lax.Precision.HIGHEST


def _dot(a, b):
    return lax.dot_general(a, b, (((1,), (0,)), ((), ())),
                           precision=_P_HIGH, preferred_element_type=jnp.float32)


def _front_body(x_ref, wp1, bp1, wp2, bp2, wp3, bp3, pars, w1, b1, wc0, bc0,
                logits_ref, q_ref, f0_ref):
    xb = x_ref[...]
    hp = jnp.maximum(_dot(xb, wp1[...]) + bp1[...][None, :], 0.0)
    hp = jnp.maximum(_dot(hp, wp2[...]) + bp2[...][None, :], 0.0)
    logits = _dot(hp, wp3[...]) + bp3[...][None, :]
    logits_ref[...] = logits
    parsing = jnp.maximum(SCALING * pars[...], 0.0)
    q_ref[...] = _dot(logits, parsing)
    h1 = _dot(xb, w1[...]) + b1[...][None, :]
    f0 = _dot(h1, wc0[...]) + bc0[...][None, :]
    f0_ref[0] = f0[:, :64]
    f0_ref[1] = f0[:, 64:]


def _front_call(x_p, Wp1, bp1, Wp2, bp2, Wp3, bp3, parsing0, W1, b1, Wc0, bc0):
    blk = 640
    grid = N_PAD // blk
    full = lambda shape: pl.BlockSpec(shape, lambda i: (0,) * len(shape))
    return pl.pallas_call(
        _front_body,
        grid=(grid,),
        in_specs=[
            pl.BlockSpec((blk, D_IN), lambda i: (i, 0)),
            full((D_IN, 512)), full((512,)),
            full((512, 64)), full((64,)),
            full((64, D_OUT)), full((D_OUT,)),
            full((D_OUT, D_OUT)),
            full((D_IN, D_HID)), full((D_HID,)),
            full((D_HID, D_HID)), full((D_HID,)),
        ],
        out_specs=[
            pl.BlockSpec((blk, D_OUT), lambda i: (i, 0)),
            pl.BlockSpec((blk, D_OUT), lambda i: (i, 0)),
            pl.BlockSpec((2, blk, 64), lambda i: (0, i, 0)),
        ],
        out_shape=[
            jax.ShapeDtypeStruct((N_PAD, D_OUT), jnp.float32),
            jax.ShapeDtypeStruct((N_PAD, D_OUT), jnp.float32),
            jax.ShapeDtypeStruct((2, N_PAD, 64), jnp.float32),
        ],
    )(x_p, Wp1, bp1, Wp2, bp2, Wp3, bp3, parsing0, W1, b1, Wc0, bc0)


def _stats_body(ew_ref, ab_ref):
    ew = ew_ref[...]
    s1 = jnp.sum(ew)
    mean = s1 / E
    var = jnp.sum((ew - mean) ** 2) / (E - 1)
    a = jnp.sqrt(1e-4 / var)
    b = 1.0 - a * mean
    col = lax.broadcasted_iota(jnp.int32, (1, 128), 1)
    ab_ref[...] = jnp.where(col == 0, a, jnp.where(col == 1, b, 0.0))


def _stats_call(ew_valid):
    return pl.pallas_call(
        _stats_body,
        out_shape=jax.ShapeDtypeStruct((1, 128), jnp.float32),
    )(ew_valid)


def _finalize_body(degp_ref, cntp_ref, ab_ref, dis_ref, alpha_ref, beta_ref):
    a = ab_ref[0, 0]
    b = ab_ref[0, 1]
    deg = a * (degp_ref[0] + degp_ref[1]) + b * (cntp_ref[0] + cntp_ref[1])
    good = deg > 1e-6
    deg_c = jnp.maximum(deg, 1e-6)
    dis = jnp.where(good, lax.rsqrt(deg_c), 0.0)
    den = MU + jnp.where(good, 1.0, 0.0)
    alpha = 1.0 / den
    dis_ref[...] = dis
    alpha_ref[...] = alpha
    beta_ref[...] = MU * alpha


def _finalize_call(degp, cntp, ab):
    return pl.pallas_call(
        _finalize_body,
        out_shape=[
            jax.ShapeDtypeStruct((80, 128), jnp.float32),
            jax.ShapeDtypeStruct((80, 128), jnp.float32),
            jax.ShapeDtypeStruct((80, 128), jnp.float32),
        ],
    )(degp, cntp, ab)


def _combine_body(p0_ref, p1_ref, f0_ref, alpha_ref, beta_ref, out_ref):
    out_ref[...] = (alpha_ref[...] * (p0_ref[...] + p1_ref[...])
                    + beta_ref[...] * f0_ref[...])


def _combine_split_body(p_ref, f0_ref, alpha_ref, beta_ref, out_ref):
    al = alpha_ref[...][None, :, :]
    be = beta_ref[...][None, :, :]
    out_ref[...] = al * p_ref[...] + be * f0_ref[...]


def _combine_split_call(p, f0h, alpha_c, beta_c):
    blk = 640
    return pl.pallas_call(
        _combine_split_body,
        grid=(N_PAD // blk,),
        in_specs=[
            pl.BlockSpec((2, blk, 64), lambda i: (0, i, 0)),
            pl.BlockSpec((2, blk, 64), lambda i: (0, i, 0)),
            pl.BlockSpec((blk, 1), lambda i: (i, 0)),
            pl.BlockSpec((blk, 1), lambda i: (i, 0)),
        ],
        out_specs=pl.BlockSpec((2, blk, 64), lambda i: (0, i, 0)),
        out_shape=jax.ShapeDtypeStruct((2, N_PAD, 64), jnp.float32),
    )(p, f0h, alpha_c, beta_c)


def _combine_call(p0, p1, f0, alpha_c, beta_c, d):
    blk = 640
    return pl.pallas_call(
        _combine_body,
        grid=(N_PAD // blk,),
        in_specs=[
            pl.BlockSpec((blk, d), lambda i: (i, 0)),
            pl.BlockSpec((blk, d), lambda i: (i, 0)),
            pl.BlockSpec((blk, d), lambda i: (i, 0)),
            pl.BlockSpec((blk, 1), lambda i: (i, 0)),
            pl.BlockSpec((blk, 1), lambda i: (i, 0)),
        ],
        out_specs=pl.BlockSpec((blk, d), lambda i: (i, 0)),
        out_shape=jax.ShapeDtypeStruct((N_PAD, d), jnp.float32),
    )(p0, p1, f0, alpha_c, beta_c)


def _comb_mm_body(p_ref, f0_ref, alpha_ref, beta_ref, wc1, bc1, out_ref):
    al = alpha_ref[...][None, :, :]
    be = beta_ref[...][None, :, :]
    f2 = al * p_ref[...] + be * f0_ref[...]
    h2 = jnp.maximum(f2, 0.0)
    w = wc1[...]
    out_ref[...] = (_dot(h2[0], w[:64]) + _dot(h2[1], w[64:])
                    + bc1[...][None, :])


def _comb_mm_call(p, f0h, alpha_c, beta_c, Wc1, bc1):
    blk = 640
    return pl.pallas_call(
        _comb_mm_body,
        grid=(N_PAD // blk,),
        in_specs=[
            pl.BlockSpec((2, blk, 64), lambda i: (0, i, 0)),
            pl.BlockSpec((2, blk, 64), lambda i: (0, i, 0)),
            pl.BlockSpec((blk, 1), lambda i: (i, 0)),
            pl.BlockSpec((blk, 1), lambda i: (i, 0)),
            pl.BlockSpec((D_HID, D_OUT), lambda i: (0, 0)),
            pl.BlockSpec((D_OUT,), lambda i: (0,)),
        ],
        out_specs=pl.BlockSpec((blk, D_OUT), lambda i: (i, 0)),
        out_shape=jax.ShapeDtypeStruct((N_PAD, D_OUT), jnp.float32),
    )(p, f0h, alpha_c, beta_c, Wc1, bc1)


_MESH = plsc.VectorSubcoreMesh(core_axis_name="c", subcore_axis_name="s",
                               num_cores=NC, num_subcores=NS)

_SC_PARAMS = pltpu.CompilerParams()
if "needs_layout_passes" in pltpu.CompilerParams.__dataclass_fields__:
    _SC_PARAMS = dataclasses.replace(_SC_PARAMS, needs_layout_passes=False)
if "use_tc_tiling_on_sc" in pltpu.CompilerParams.__dataclass_fields__:
    _SC_PARAMS = dataclasses.replace(_SC_PARAMS, use_tc_tiling_on_sc=False)


def _iota16():
    return lax.iota(jnp.int32, LL)


def _zero_vec_ref(ref, n):
    z = jnp.zeros((LL,), jnp.float32)

    @pl.loop(0, n // LL)
    def _(i):
        ref[pl.ds(i * LL, LL)] = z


def _ew_deg_body(logits_hbm, q_hbm, row_hbm, col_hbm,
                 ew_hbm, degp_hbm, cntp_hbm,
                 ridx, cidx, abuf, bbuf, ewbuf, obuf, zbuf,
                 deg_sh, cnt_sh):
    cid = lax.axis_index("c")
    sid = lax.axis_index("s")
    wid = cid * NS + sid
    base = wid * NCH

    pltpu.sync_copy(row_hbm.at[pl.ds(base, NCH)], ridx)
    pltpu.sync_copy(col_hbm.at[pl.ds(base, NCH)], cidx)

    _zero_vec_ref(zbuf, NPT)
    pltpu.sync_copy(zbuf, deg_sh.at[pl.ds(sid * NPT, NPT)])
    pltpu.sync_copy(zbuf, cnt_sh.at[pl.ds(sid * NPT, NPT)])

    ones = jnp.ones((LL,), jnp.float32)

    @pl.loop(0, ECH // LL)
    def _(i):
        obuf[pl.ds(i * LL, LL)] = ones

    plsc.subcore_barrier()

    @pl.loop(0, NCH)
    def _(j):
        rj = ridx.at[j]
        cj = cidx.at[j]
        pltpu.sync_copy(logits_hbm.at[rj], abuf)
        pltpu.sync_copy(q_hbm.at[cj], bbuf)
        for g in range(ECH // LL):
            erow = g * LL + _iota16()
            acc = jnp.zeros((LL,), jnp.float32)
            for c in range(D_OUT):
                fcol = jnp.full((LL,), c, jnp.int32)
                av = plsc.load_gather(abuf, [erow, fcol])
                bv = plsc.load_gather(bbuf, [erow, fcol])
                acc = acc + av * bv
            ewbuf[pl.ds(g * LL, LL)] = acc
        pltpu.sync_copy(ewbuf, ew_hbm.at[base + j])
        pltpu.sync_copy(ewbuf, deg_sh.at[rj], add=True)
        pltpu.sync_copy(obuf, cnt_sh.at[rj], add=True)

    plsc.subcore_barrier()
    pltpu.sync_copy(deg_sh.at[pl.ds(sid * NPT, NPT)],
                    degp_hbm.at[cid, pl.ds(sid * NPT, NPT)])
    pltpu.sync_copy(cnt_sh.at[pl.ds(sid * NPT, NPT)],
                    cntp_hbm.at[cid, pl.ds(sid * NPT, NPT)])


def _ew_deg_call(logits_p, q_p, row2d, col2d):
    kern = pl.kernel(
        _ew_deg_body,
        out_type=[
            jax.ShapeDtypeStruct((E_PAD // ECH, ECH), jnp.float32),
            jax.ShapeDtypeStruct((NC, N_PAD), jnp.float32),
            jax.ShapeDtypeStruct((NC, N_PAD), jnp.float32),
        ],
        mesh=_MESH,
        compiler_params=_SC_PARAMS,
        scratch_types=[
            pltpu.VMEM((NCH, ECH), jnp.int32),
            pltpu.VMEM((NCH, ECH), jnp.int32),
            pltpu.VMEM((ECH, D_OUT), jnp.float32),
            pltpu.VMEM((ECH, D_OUT), jnp.float32),
            pltpu.VMEM((ECH,), jnp.float32),
            pltpu.VMEM((ECH,), jnp.float32),
            pltpu.VMEM((NPT,), jnp.float32),
            pltpu.VMEM_SHARED((N_PAD,), jnp.float32),
            pltpu.VMEM_SHARED((N_PAD,), jnp.float32),
        ],
    )
    return kern(logits_p, q_p, row2d, col2d)


def _coef_body(ew_hbm, row_hbm, col_hbm, dis_hbm, a_hbm, b_hbm, coef_hbm,
               ridx, cidx, ewbuf, cfbuf, disv, avr, bvr):
    cid = lax.axis_index("c")
    sid = lax.axis_index("s")
    wid = cid * NS + sid
    base = wid * NCH

    pltpu.sync_copy(row_hbm.at[pl.ds(base, NCH)], ridx)
    pltpu.sync_copy(col_hbm.at[pl.ds(base, NCH)], cidx)
    pltpu.sync_copy(dis_hbm, disv)
    pltpu.sync_copy(a_hbm, avr)
    pltpu.sync_copy(b_hbm, bvr)
    av = avr[...]
    bv = bvr[...]

    @pl.loop(0, NCH)
    def _(j):
        pltpu.sync_copy(ew_hbm.at[base + j], ewbuf)
        for g in range(ECH // LL):
            sl = pl.ds(g * LL, LL)
            rv = ridx[j, sl]
            cv = cidx[j, sl]
            dr = plsc.load_gather(disv, [rv])
            dc = plsc.load_gather(disv, [cv])
            cfbuf[sl] = (av * ewbuf[sl] + bv) * dr * dc
        pltpu.sync_copy(cfbuf, coef_hbm.at[base + j])


def _coef_call(ew2d, row2d, col2d, dis_flat, a16, b16):
    kern = pl.kernel(
        _coef_body,
        out_type=jax.ShapeDtypeStruct((E_PAD // ECH, ECH), jnp.float32),
        mesh=_MESH,
        compiler_params=_SC_PARAMS,
        scratch_types=[
            pltpu.VMEM((NCH, ECH), jnp.int32),
            pltpu.VMEM((NCH, ECH), jnp.int32),
            pltpu.VMEM((ECH,), jnp.float32),
            pltpu.VMEM((ECH,), jnp.float32),
            pltpu.VMEM((N_PAD,), jnp.float32),
            pltpu.VMEM((LL,), jnp.float32),
            pltpu.VMEM((LL,), jnp.float32),
        ],
    )
    return kern(ew2d, row2d, col2d, dis_flat, a16, b16)


def _spmv_body(d, nb, nch, split, f_hbm, row_hbm, col_hbm, coef_hbm, aggp_hbm,
               ridx, cidx, cvm, rows, agg_sh, *sems):
    gsems = sems[:nb]
    ssems = sems[nb:]
    pd = nb // 2
    cid = lax.axis_index("c")
    sid = lax.axis_index("s")
    if split:
        base = sid * nch
        table = f_hbm.at[cid]
    else:
        base = (cid * NS + sid) * nch
        table = f_hbm

    pltpu.sync_copy(row_hbm.at[pl.ds(base, nch)], ridx)
    pltpu.sync_copy(col_hbm.at[pl.ds(base, nch)], cidx)
    pltpu.sync_copy(coef_hbm.at[pl.ds(base, nch)], cvm)

    z = jnp.zeros((LL,), jnp.float32)

    @pl.loop(0, 64)
    def _(i):
        for jj in range(d // LL):
            rows[i, pl.ds(jj * LL, LL)] = z

    zsrc = rows.at[pl.ds(0, 64)]

    @pl.loop(0, NPT // 64)
    def _(k):
        pltpu.sync_copy(zsrc, agg_sh.at[pl.ds(sid * NPT + k * 64, 64)])

    plsc.subcore_barrier()

    def rows_at(b):
        return rows.at[pl.ds(b * ECH, ECH)]

    def issue_gather(k, b):
        pltpu.async_copy(table.at[cidx.at[k]], rows_at(b), gsems[b])

    def wait_gather(k, b):
        pltpu.make_async_copy(table.at[cidx.at[k]], rows_at(b), gsems[b]).wait()

    def issue_scatter(k, b):
        pltpu.async_copy(rows_at(b), agg_sh.at[ridx.at[k]], ssems[b], add=True)

    def wait_scatter(k, b):
        pltpu.make_async_copy(rows_at(b), agg_sh.at[ridx.at[k]],
                              ssems[b]).wait()

    def scale(k, b):
        kv = jnp.full((LL,), k, jnp.int32)

        @pl.loop(0, ECH, step=2)
        def _(e):
            e1 = e + 1
            cb0 = plsc.load_gather(cvm, [kv, jnp.full((LL,), e, jnp.int32)])
            cb1 = plsc.load_gather(cvm, [kv, jnp.full((LL,), e1, jnp.int32)])
            r = rows_at(b)
            for jj in range(d // LL):
                sl = pl.ds(jj * LL, LL)
                r[e, sl] = r[e, sl] * cb0
            for jj in range(d // LL):
                sl = pl.ds(jj * LL, LL)
                r[e1, sl] = r[e1, sl] * cb1

    for k in range(pd):
        issue_gather(k, k)

    @pl.loop(0, nch // nb)
    def _(outer):
        k0 = outer * nb
        for i in range(nb):
            k = k0 + i
            wait_gather(k, i)
            bp = (i + pd) % nb
            kp = k + pd
            kold = kp - nb

            @pl.when(kp < nch)
            def _():
                @pl.when(kold >= 0)
                def _():
                    wait_scatter(kold, bp)
                issue_gather(kp, bp)

            scale(k, i)
            issue_scatter(k, i)

    for k in range(nch - nb, nch):
        wait_scatter(k, k % nb)

    plsc.subcore_barrier()
    pltpu.sync_copy(agg_sh.at[pl.ds(sid * NPT, NPT)],
                    aggp_hbm.at[cid, pl.ds(sid * NPT, NPT)])


@functools.cache
def _spmv_kernel(d):
    split = d == 64
    nb = 4
    nch = NCH * NC if split else NCH
    sems = [pltpu.SemaphoreType.DMA] * (2 * nb)
    out_t = jax.ShapeDtypeStruct((NC, N_PAD, d), jnp.float32)
    return pl.kernel(
        functools.partial(_spmv_body, d, nb, nch, split),
        out_type=out_t,
        mesh=_MESH,
        compiler_params=_SC_PARAMS,
        scratch_types=[
            pltpu.VMEM((nch, ECH), jnp.int32),
            pltpu.VMEM((nch, ECH), jnp.int32),
            pltpu.VMEM((nch, ECH), jnp.float32),
            pltpu.VMEM((nb * ECH, d), jnp.float32),
            pltpu.VMEM_SHARED((N_PAD, d), jnp.float32),
        ] + sems,
    )


def _spmv_call(f, row2d, col2d, coef2d, d):
    return _spmv_kernel(d)(f, row2d, col2d, coef2d)


def kernel(x, edge_index, W1, b1, Wp1, bp1, Wp2, bp2, Wp3, bp3, parsing0,
           Wc0, bc0, Wc1, bc1):
    x_p = jnp.pad(x, ((0, N_PAD - N), (0, 0)))
    row = edge_index[0]
    col = edge_index[1]
    row_p = jnp.concatenate(
        [row, jnp.full((E_PAD - E,), N, jnp.int32)]).reshape(E_PAD // ECH, ECH)
    col_p = jnp.concatenate(
        [col, jnp.zeros((E_PAD - E,), jnp.int32)]).reshape(E_PAD // ECH, ECH)

    logits_p, q_p, f0h = _front_call(
        x_p, Wp1, bp1, Wp2, bp2, Wp3, bp3, parsing0, W1, b1, Wc0, bc0)

    ew2d, degp, cntp = _ew_deg_call(logits_p, q_p, row_p, col_p)

    ew_valid = ew2d.reshape(-1)[:E].reshape(E // 128, 128)
    ab = _stats_call(ew_valid)

    dis, alpha, beta = _finalize_call(
        degp.reshape(NC, N_PAD // 128, 128),
        cntp.reshape(NC, N_PAD // 128, 128), ab)

    dis_flat = dis.reshape(N_PAD)
    a16 = jnp.broadcast_to(ab.reshape(128)[0], (LL,))
    b16 = jnp.broadcast_to(ab.reshape(128)[1], (LL,))
    coef2d = _coef_call(ew2d, row_p, col_p, dis_flat, a16, b16)

    alpha_c = alpha.reshape(N_PAD, 1)
    beta_c = beta.reshape(N_PAD, 1)

    p = _spmv_call(f0h, row_p, col_p, coef2d, 64)
    f1h = _combine_split_call(p, f0h, alpha_c, beta_c)
    p = _spmv_call(f1h, row_p, col_p, coef2d, 64)
    g0 = _comb_mm_call(p, f0h, alpha_c, beta_c, Wc1, bc1)

    p = _spmv_call(g0, row_p, col_p, coef2d, D_OUT)
    g1 = _combine_call(p[0], p[1], g0, alpha_c, beta_c, D_OUT)
    p = _spmv_call(g1, row_p, col_p, coef2d, D_OUT)
    g2 = _combine_call(p[0], p[1], g0, alpha_c, beta_c, D_OUT)

    return g2[:N]


# speedup vs baseline: 8.1269x; 1.0968x over previous
import dataclasses
import functools

import jax
import jax.numpy as jnp
from jax import lax
from jax.experimental import pallas as pl
from jax.experimental.pallas import tpu as pltpu
from jax.experimental.pallas import tpu_sc as plsc

N = 10000
E = 160000
D_IN, D_HID, D_OUT = 256, 128, 16
MU = 0.1
SCALING = 2.0

NC, NS, LL = 2, 16, 16
NW = NC * NS
N_PAD = 10240
E_PAD = 163840
EPT = E_PAD // NW
ECH = 128
NCH = EPT // ECH
NPT = N_PAD // NS

_P_HIGH = lax.Precision.HIGHEST


def _dot(a, b):
    return lax.dot_general(a, b, (((1,), (0,)), ((), ())),
                           precision=_P_HIGH, preferred_element_type=jnp.float32)


def _front_body(x_ref, wp1, bp1, wp2, bp2, wp3, bp3, pars, w1, b1, wc0, bc0,
                logits_ref, q_ref, f0_ref):
    xb = x_ref[...]
    hp = jnp.maximum(_dot(xb, wp1[...]) + bp1[...][None, :], 0.0)
    hp = jnp.maximum(_dot(hp, wp2[...]) + bp2[...][None, :], 0.0)
    logits = _dot(hp, wp3[...]) + bp3[...][None, :]
    logits_ref[...] = logits
    parsing = jnp.maximum(SCALING * pars[...], 0.0)
    q_ref[...] = _dot(logits, parsing)
    h1 = _dot(xb, w1[...]) + b1[...][None, :]
    f0 = _dot(h1, wc0[...]) + bc0[...][None, :]
    f0_ref[0] = f0[:, :64]
    f0_ref[1] = f0[:, 64:]


def _front_call(x_p, Wp1, bp1, Wp2, bp2, Wp3, bp3, parsing0, W1, b1, Wc0, bc0):
    blk = 640
    grid = N_PAD // blk
    full = lambda shape: pl.BlockSpec(shape, lambda i: (0,) * len(shape))
    return pl.pallas_call(
        _front_body,
        grid=(grid,),
        in_specs=[
            pl.BlockSpec((blk, D_IN), lambda i: (i, 0)),
            full((D_IN, 512)), full((512,)),
            full((512, 64)), full((64,)),
            full((64, D_OUT)), full((D_OUT,)),
            full((D_OUT, D_OUT)),
            full((D_IN, D_HID)), full((D_HID,)),
            full((D_HID, D_HID)), full((D_HID,)),
        ],
        out_specs=[
            pl.BlockSpec((blk, D_OUT), lambda i: (i, 0)),
            pl.BlockSpec((blk, D_OUT), lambda i: (i, 0)),
            pl.BlockSpec((2, blk, 64), lambda i: (0, i, 0)),
        ],
        out_shape=[
            jax.ShapeDtypeStruct((N_PAD, D_OUT), jnp.float32),
            jax.ShapeDtypeStruct((N_PAD, D_OUT), jnp.float32),
            jax.ShapeDtypeStruct((2, N_PAD, 64), jnp.float32),
        ],
    )(x_p, Wp1, bp1, Wp2, bp2, Wp3, bp3, parsing0, W1, b1, Wc0, bc0)


def _stats_body(ew_ref, ab_ref):
    ew = ew_ref[...]
    s1 = jnp.sum(ew)
    mean = s1 / E
    var = jnp.sum((ew - mean) ** 2) / (E - 1)
    a = jnp.sqrt(1e-4 / var)
    b = 1.0 - a * mean
    col = lax.broadcasted_iota(jnp.int32, (1, 128), 1)
    ab_ref[...] = jnp.where(col == 0, a, jnp.where(col == 1, b, 0.0))


def _stats_call(ew_valid):
    return pl.pallas_call(
        _stats_body,
        out_shape=jax.ShapeDtypeStruct((1, 128), jnp.float32),
    )(ew_valid)


def _finalize_body(degp_ref, cntp_ref, ab_ref, dis_ref, alpha_ref, beta_ref):
    a = ab_ref[0, 0]
    b = ab_ref[0, 1]
    deg = a * (degp_ref[0] + degp_ref[1]) + b * (cntp_ref[0] + cntp_ref[1])
    good = deg > 1e-6
    deg_c = jnp.maximum(deg, 1e-6)
    dis = jnp.where(good, lax.rsqrt(deg_c), 0.0)
    den = MU + jnp.where(good, 1.0, 0.0)
    alpha = 1.0 / den
    dis_ref[...] = dis
    alpha_ref[...] = alpha
    beta_ref[...] = MU * alpha


def _finalize_call(degp, cntp, ab):
    return pl.pallas_call(
        _finalize_body,
        out_shape=[
            jax.ShapeDtypeStruct((80, 128), jnp.float32),
            jax.ShapeDtypeStruct((80, 128), jnp.float32),
            jax.ShapeDtypeStruct((80, 128), jnp.float32),
        ],
    )(degp, cntp, ab)


def _combine_body(p0_ref, p1_ref, f0_ref, alpha_ref, beta_ref, out_ref):
    out_ref[...] = (alpha_ref[...] * (p0_ref[...] + p1_ref[...])
                    + beta_ref[...] * f0_ref[...])


def _combine_split_body(p_ref, f0_ref, alpha_ref, beta_ref, out_ref):
    al = alpha_ref[...][None, :, :]
    be = beta_ref[...][None, :, :]
    out_ref[...] = al * p_ref[...] + be * f0_ref[...]


def _combine_split_call(p, f0h, alpha_c, beta_c):
    blk = 640
    return pl.pallas_call(
        _combine_split_body,
        grid=(N_PAD // blk,),
        in_specs=[
            pl.BlockSpec((2, blk, 64), lambda i: (0, i, 0)),
            pl.BlockSpec((2, blk, 64), lambda i: (0, i, 0)),
            pl.BlockSpec((blk, 1), lambda i: (i, 0)),
            pl.BlockSpec((blk, 1), lambda i: (i, 0)),
        ],
        out_specs=pl.BlockSpec((2, blk, 64), lambda i: (0, i, 0)),
        out_shape=jax.ShapeDtypeStruct((2, N_PAD, 64), jnp.float32),
    )(p, f0h, alpha_c, beta_c)


def _combine_call(p0, p1, f0, alpha_c, beta_c, d):
    blk = 640
    return pl.pallas_call(
        _combine_body,
        grid=(N_PAD // blk,),
        in_specs=[
            pl.BlockSpec((blk, d), lambda i: (i, 0)),
            pl.BlockSpec((blk, d), lambda i: (i, 0)),
            pl.BlockSpec((blk, d), lambda i: (i, 0)),
            pl.BlockSpec((blk, 1), lambda i: (i, 0)),
            pl.BlockSpec((blk, 1), lambda i: (i, 0)),
        ],
        out_specs=pl.BlockSpec((blk, d), lambda i: (i, 0)),
        out_shape=jax.ShapeDtypeStruct((N_PAD, d), jnp.float32),
    )(p0, p1, f0, alpha_c, beta_c)


def _comb_mm_body(p_ref, f0_ref, alpha_ref, beta_ref, wc1, bc1, out_ref):
    al = alpha_ref[...][None, :, :]
    be = beta_ref[...][None, :, :]
    f2 = al * p_ref[...] + be * f0_ref[...]
    h2 = jnp.maximum(f2, 0.0)
    w = wc1[...]
    out_ref[...] = (_dot(h2[0], w[:64]) + _dot(h2[1], w[64:])
                    + bc1[...][None, :])


def _comb_mm_call(p, f0h, alpha_c, beta_c, Wc1, bc1):
    blk = 640
    return pl.pallas_call(
        _comb_mm_body,
        grid=(N_PAD // blk,),
        in_specs=[
            pl.BlockSpec((2, blk, 64), lambda i: (0, i, 0)),
            pl.BlockSpec((2, blk, 64), lambda i: (0, i, 0)),
            pl.BlockSpec((blk, 1), lambda i: (i, 0)),
            pl.BlockSpec((blk, 1), lambda i: (i, 0)),
            pl.BlockSpec((D_HID, D_OUT), lambda i: (0, 0)),
            pl.BlockSpec((D_OUT,), lambda i: (0,)),
        ],
        out_specs=pl.BlockSpec((blk, D_OUT), lambda i: (i, 0)),
        out_shape=jax.ShapeDtypeStruct((N_PAD, D_OUT), jnp.float32),
    )(p, f0h, alpha_c, beta_c, Wc1, bc1)


_MESH = plsc.VectorSubcoreMesh(core_axis_name="c", subcore_axis_name="s",
                               num_cores=NC, num_subcores=NS)

_SC_PARAMS = pltpu.CompilerParams()
if "needs_layout_passes" in pltpu.CompilerParams.__dataclass_fields__:
    _SC_PARAMS = dataclasses.replace(_SC_PARAMS, needs_layout_passes=False)
if "use_tc_tiling_on_sc" in pltpu.CompilerParams.__dataclass_fields__:
    _SC_PARAMS = dataclasses.replace(_SC_PARAMS, use_tc_tiling_on_sc=False)


def _iota16():
    return lax.iota(jnp.int32, LL)


def _zero_vec_ref(ref, n):
    z = jnp.zeros((LL,), jnp.float32)

    @pl.loop(0, n // LL)
    def _(i):
        ref[pl.ds(i * LL, LL)] = z


def _ew_deg_body(logits_hbm, q_hbm, row_hbm, col_hbm,
                 ew_hbm, degp_hbm, cntp_hbm,
                 ridx, cidx, abuf, bbuf, ewbuf, obuf, zbuf,
                 deg_sh, cnt_sh, gsA, gsB, wsem, dsem, csem):
    cid = lax.axis_index("c")
    sid = lax.axis_index("s")
    wid = cid * NS + sid
    base = wid * NCH

    pltpu.sync_copy(row_hbm.at[pl.ds(base, NCH)], ridx)
    pltpu.sync_copy(col_hbm.at[pl.ds(base, NCH)], cidx)

    _zero_vec_ref(zbuf, NPT)
    pltpu.sync_copy(zbuf, deg_sh.at[pl.ds(sid * NPT, NPT)])
    pltpu.sync_copy(zbuf, cnt_sh.at[pl.ds(sid * NPT, NPT)])

    ones = jnp.ones((LL,), jnp.float32)

    @pl.loop(0, ECH // LL)
    def _(i):
        obuf[pl.ds(i * LL, LL)] = ones

    plsc.subcore_barrier()

    def abuf_at(b):
        return abuf.at[pl.ds(b * ECH, ECH)]

    def bbuf_at(b):
        return bbuf.at[pl.ds(b * ECH, ECH)]

    def ewb_at(b):
        return ewbuf.at[pl.ds(b * ECH, ECH)]

    def issue_gathers(k, b):
        pltpu.async_copy(logits_hbm.at[ridx.at[k]], abuf_at(b), gsA)
        pltpu.async_copy(q_hbm.at[cidx.at[k]], bbuf_at(b), gsB)

    def wait_gathers(k, b):
        pltpu.make_async_copy(logits_hbm.at[ridx.at[k]], abuf_at(b), gsA).wait()
        pltpu.make_async_copy(q_hbm.at[cidx.at[k]], bbuf_at(b), gsB).wait()

    def issue_outs(k, b):
        pltpu.async_copy(ewb_at(b), ew_hbm.at[base + k], wsem)
        pltpu.async_copy(ewb_at(b), deg_sh.at[ridx.at[k]], dsem, add=True)
        pltpu.async_copy(obuf, cnt_sh.at[ridx.at[k]], csem, add=True)

    def wait_outs(k, b):
        pltpu.make_async_copy(ewb_at(b), ew_hbm.at[base + k], wsem).wait()
        pltpu.make_async_copy(ewb_at(b), deg_sh.at[ridx.at[k]], dsem).wait()

    issue_gathers(0, 0)

    @pl.loop(0, NCH // 2)
    def _(outer):
        k0 = outer * 2
        for i in range(2):
            k = k0 + i
            b = i
            wait_gathers(k, b)

            @pl.when(k + 1 < NCH)
            def _():
                issue_gathers(k + 1, 1 - b)

            @pl.when(k >= 2)
            def _():
                wait_outs(k - 2, b)

            ab = abuf_at(b)
            bb = bbuf_at(b)
            eb = ewb_at(b)
            for g in range(ECH // LL):
                erow = g * LL + _iota16()
                acc = jnp.zeros((LL,), jnp.float32)
                for c in range(D_OUT):
                    fcol = jnp.full((LL,), c, jnp.int32)
                    av = plsc.load_gather(ab, [erow, fcol])
                    bv = plsc.load_gather(bb, [erow, fcol])
                    acc = acc + av * bv
                eb[pl.ds(g * LL, LL)] = acc
            issue_outs(k, b)

    for k in range(NCH - 2, NCH):
        wait_outs(k, k % 2)

    @pl.loop(0, NCH)
    def _(k):
        pltpu.make_async_copy(obuf, cnt_sh.at[ridx.at[k]], csem).wait()

    plsc.subcore_barrier()
    pltpu.sync_copy(deg_sh.at[pl.ds(sid * NPT, NPT)],
                    degp_hbm.at[cid, pl.ds(sid * NPT, NPT)])
    pltpu.sync_copy(cnt_sh.at[pl.ds(sid * NPT, NPT)],
                    cntp_hbm.at[cid, pl.ds(sid * NPT, NPT)])


def _ew_deg_call(logits_p, q_p, row2d, col2d):
    kern = pl.kernel(
        _ew_deg_body,
        out_type=[
            jax.ShapeDtypeStruct((E_PAD // ECH, ECH), jnp.float32),
            jax.ShapeDtypeStruct((NC, N_PAD), jnp.float32),
            jax.ShapeDtypeStruct((NC, N_PAD), jnp.float32),
        ],
        mesh=_MESH,
        compiler_params=_SC_PARAMS,
        scratch_types=[
            pltpu.VMEM((NCH, ECH), jnp.int32),
            pltpu.VMEM((NCH, ECH), jnp.int32),
            pltpu.VMEM((2 * ECH, D_OUT), jnp.float32),
            pltpu.VMEM((2 * ECH, D_OUT), jnp.float32),
            pltpu.VMEM((2 * ECH,), jnp.float32),
            pltpu.VMEM((ECH,), jnp.float32),
            pltpu.VMEM((NPT,), jnp.float32),
            pltpu.VMEM_SHARED((N_PAD,), jnp.float32),
            pltpu.VMEM_SHARED((N_PAD,), jnp.float32),
            pltpu.SemaphoreType.DMA, pltpu.SemaphoreType.DMA,
            pltpu.SemaphoreType.DMA, pltpu.SemaphoreType.DMA,
            pltpu.SemaphoreType.DMA,
        ],
    )
    return kern(logits_p, q_p, row2d, col2d)


def _coef_body(ew_hbm, row_hbm, col_hbm, dis_hbm, a_hbm, b_hbm, coef_hbm,
               ridx, cidx, ewbuf, cfbuf, disv, avr, bvr):
    cid = lax.axis_index("c")
    sid = lax.axis_index("s")
    wid = cid * NS + sid
    base = wid * NCH

    pltpu.sync_copy(row_hbm.at[pl.ds(base, NCH)], ridx)
    pltpu.sync_copy(col_hbm.at[pl.ds(base, NCH)], cidx)
    pltpu.sync_copy(dis_hbm, disv)
    pltpu.sync_copy(a_hbm, avr)
    pltpu.sync_copy(b_hbm, bvr)
    av = avr[...]
    bv = bvr[...]

    @pl.loop(0, NCH)
    def _(j):
        pltpu.sync_copy(ew_hbm.at[base + j], ewbuf)
        for g in range(ECH // LL):
            sl = pl.ds(g * LL, LL)
            rv = ridx[j, sl]
            cv = cidx[j, sl]
            dr = plsc.load_gather(disv, [rv])
            dc = plsc.load_gather(disv, [cv])
            cfbuf[sl] = (av * ewbuf[sl] + bv) * dr * dc
        pltpu.sync_copy(cfbuf, coef_hbm.at[base + j])


def _coef_call(ew2d, row2d, col2d, dis_flat, a16, b16):
    kern = pl.kernel(
        _coef_body,
        out_type=jax.ShapeDtypeStruct((E_PAD // ECH, ECH), jnp.float32),
        mesh=_MESH,
        compiler_params=_SC_PARAMS,
        scratch_types=[
            pltpu.VMEM((NCH, ECH), jnp.int32),
            pltpu.VMEM((NCH, ECH), jnp.int32),
            pltpu.VMEM((ECH,), jnp.float32),
            pltpu.VMEM((ECH,), jnp.float32),
            pltpu.VMEM((N_PAD,), jnp.float32),
            pltpu.VMEM((LL,), jnp.float32),
            pltpu.VMEM((LL,), jnp.float32),
        ],
    )
    return kern(ew2d, row2d, col2d, dis_flat, a16, b16)


def _spmv_body(d, nb, nch, split, f_hbm, row_hbm, col_hbm, coef_hbm, aggp_hbm,
               ridx, cidx, cvm, rows, agg_sh, *sems):
    gsems = sems[:nb]
    ssems = sems[nb:]
    pd = nb // 2
    cid = lax.axis_index("c")
    sid = lax.axis_index("s")
    if split:
        base = sid * nch
        table = f_hbm.at[cid]
    else:
        base = (cid * NS + sid) * nch
        table = f_hbm

    pltpu.sync_copy(row_hbm.at[pl.ds(base, nch)], ridx)
    pltpu.sync_copy(col_hbm.at[pl.ds(base, nch)], cidx)
    pltpu.sync_copy(coef_hbm.at[pl.ds(base, nch)], cvm)

    z = jnp.zeros((LL,), jnp.float32)

    @pl.loop(0, 64)
    def _(i):
        for jj in range(d // LL):
            rows[i, pl.ds(jj * LL, LL)] = z

    zsrc = rows.at[pl.ds(0, 64)]

    @pl.loop(0, NPT // 64)
    def _(k):
        pltpu.sync_copy(zsrc, agg_sh.at[pl.ds(sid * NPT + k * 64, 64)])

    plsc.subcore_barrier()

    def rows_at(b):
        return rows.at[pl.ds(b * ECH, ECH)]

    def issue_gather(k, b):
        pltpu.async_copy(table.at[cidx.at[k]], rows_at(b), gsems[b])

    def wait_gather(k, b):
        pltpu.make_async_copy(table.at[cidx.at[k]], rows_at(b), gsems[b]).wait()

    def issue_scatter(k, b):
        pltpu.async_copy(rows_at(b), agg_sh.at[ridx.at[k]], ssems[b], add=True)

    def wait_scatter(k, b):
        pltpu.make_async_copy(rows_at(b), agg_sh.at[ridx.at[k]],
                              ssems[b]).wait()

    def scale(k, b):
        kv = jnp.full((LL,), k, jnp.int32)

        @pl.loop(0, ECH, step=2)
        def _(e):
            e1 = e + 1
            cb0 = plsc.load_gather(cvm, [kv, jnp.full((LL,), e, jnp.int32)])
            cb1 = plsc.load_gather(cvm, [kv, jnp.full((LL,), e1, jnp.int32)])
            r = rows_at(b)
            for jj in range(d // LL):
                sl = pl.ds(jj * LL, LL)
                r[e, sl] = r[e, sl] * cb0
            for jj in range(d // LL):
                sl = pl.ds(jj * LL, LL)
                r[e1, sl] = r[e1, sl] * cb1

    for k in range(pd):
        issue_gather(k, k)

    @pl.loop(0, nch // nb)
    def _(outer):
        k0 = outer * nb
        for i in range(nb):
            k = k0 + i
            wait_gather(k, i)
            bp = (i + pd) % nb
            kp = k + pd
            kold = kp - nb

            @pl.when(kp < nch)
            def _():
                @pl.when(kold >= 0)
                def _():
                    wait_scatter(kold, bp)
                issue_gather(kp, bp)

            scale(k, i)
            issue_scatter(k, i)

    for k in range(nch - nb, nch):
        wait_scatter(k, k % nb)

    plsc.subcore_barrier()
    pltpu.sync_copy(agg_sh.at[pl.ds(sid * NPT, NPT)],
                    aggp_hbm.at[cid, pl.ds(sid * NPT, NPT)])


@functools.cache
def _spmv_kernel(d):
    split = d == 64
    nb = 4
    nch = NCH * NC if split else NCH
    sems = [pltpu.SemaphoreType.DMA] * (2 * nb)
    out_t = jax.ShapeDtypeStruct((NC, N_PAD, d), jnp.float32)
    return pl.kernel(
        functools.partial(_spmv_body, d, nb, nch, split),
        out_type=out_t,
        mesh=_MESH,
        compiler_params=_SC_PARAMS,
        scratch_types=[
            pltpu.VMEM((nch, ECH), jnp.int32),
            pltpu.VMEM((nch, ECH), jnp.int32),
            pltpu.VMEM((nch, ECH), jnp.float32),
            pltpu.VMEM((nb * ECH, d), jnp.float32),
            pltpu.VMEM_SHARED((N_PAD, d), jnp.float32),
        ] + sems,
    )


def _spmv_call(f, row2d, col2d, coef2d, d):
    return _spmv_kernel(d)(f, row2d, col2d, coef2d)


def kernel(x, edge_index, W1, b1, Wp1, bp1, Wp2, bp2, Wp3, bp3, parsing0,
           Wc0, bc0, Wc1, bc1):
    x_p = jnp.pad(x, ((0, N_PAD - N), (0, 0)))
    row = edge_index[0]
    col = edge_index[1]
    row_p = jnp.concatenate(
        [row, jnp.full((E_PAD - E,), N, jnp.int32)]).reshape(E_PAD // ECH, ECH)
    col_p = jnp.concatenate(
        [col, jnp.zeros((E_PAD - E,), jnp.int32)]).reshape(E_PAD // ECH, ECH)

    logits_p, q_p, f0h = _front_call(
        x_p, Wp1, bp1, Wp2, bp2, Wp3, bp3, parsing0, W1, b1, Wc0, bc0)

    ew2d, degp, cntp = _ew_deg_call(logits_p, q_p, row_p, col_p)

    ew_valid = ew2d.reshape(-1)[:E].reshape(E // 128, 128)
    ab = _stats_call(ew_valid)

    dis, alpha, beta = _finalize_call(
        degp.reshape(NC, N_PAD // 128, 128),
        cntp.reshape(NC, N_PAD // 128, 128), ab)

    dis_flat = dis.reshape(N_PAD)
    a16 = jnp.broadcast_to(ab.reshape(128)[0], (LL,))
    b16 = jnp.broadcast_to(ab.reshape(128)[1], (LL,))
    coef2d = _coef_call(ew2d, row_p, col_p, dis_flat, a16, b16)

    alpha_c = alpha.reshape(N_PAD, 1)
    beta_c = beta.reshape(N_PAD, 1)

    p = _spmv_call(f0h, row_p, col_p, coef2d, 64)
    f1h = _combine_split_call(p, f0h, alpha_c, beta_c)
    p = _spmv_call(f1h, row_p, col_p, coef2d, 64)
    g0 = _comb_mm_call(p, f0h, alpha_c, beta_c, Wc1, bc1)

    p = _spmv_call(g0, row_p, col_p, coef2d, D_OUT)
    g1 = _combine_call(p[0], p[1], g0, alpha_c, beta_c, D_OUT)
    p = _spmv_call(g1, row_p, col_p, coef2d, D_OUT)
    g2 = _combine_call(p[0], p[1], g0, alpha_c, beta_c, D_OUT)

    return g2[:N]


# speedup vs baseline: 8.1458x; 1.0023x over previous
import dataclasses
import functools

import jax
import jax.numpy as jnp
from jax import lax
from jax.experimental import pallas as pl
from jax.experimental.pallas import tpu as pltpu
from jax.experimental.pallas import tpu_sc as plsc

N = 10000
E = 160000
D_IN, D_HID, D_OUT = 256, 128, 16
MU = 0.1
SCALING = 2.0

NC, NS, LL = 2, 16, 16
NW = NC * NS
N_PAD = 10240
E_PAD = 163840
EPT = E_PAD // NW
ECH = 128
NCH = EPT // ECH
NPT = N_PAD // NS

_P_HIGH = lax.Precision.HIGHEST


def _dot(a, b):
    return lax.dot_general(a, b, (((1,), (0,)), ((), ())),
                           precision=_P_HIGH, preferred_element_type=jnp.float32)


def _front_body(x_ref, wp1, bp1, wp2, bp2, wp3, bp3, pars, w1, b1, wc0, bc0,
                logits_ref, q_ref, f0_ref):
    xb = x_ref[...]
    hp = jnp.maximum(_dot(xb, wp1[...]) + bp1[...][None, :], 0.0)
    hp = jnp.maximum(_dot(hp, wp2[...]) + bp2[...][None, :], 0.0)
    logits = _dot(hp, wp3[...]) + bp3[...][None, :]
    logits_ref[...] = logits
    parsing = jnp.maximum(SCALING * pars[...], 0.0)
    q_ref[...] = _dot(logits, parsing)
    h1 = _dot(xb, w1[...]) + b1[...][None, :]
    f0 = _dot(h1, wc0[...]) + bc0[...][None, :]
    f0_ref[0] = f0[:, :64]
    f0_ref[1] = f0[:, 64:]


def _front_call(x_p, Wp1, bp1, Wp2, bp2, Wp3, bp3, parsing0, W1, b1, Wc0, bc0):
    blk = 640
    grid = N_PAD // blk
    full = lambda shape: pl.BlockSpec(shape, lambda i: (0,) * len(shape))
    return pl.pallas_call(
        _front_body,
        grid=(grid,),
        in_specs=[
            pl.BlockSpec((blk, D_IN), lambda i: (i, 0)),
            full((D_IN, 512)), full((512,)),
            full((512, 64)), full((64,)),
            full((64, D_OUT)), full((D_OUT,)),
            full((D_OUT, D_OUT)),
            full((D_IN, D_HID)), full((D_HID,)),
            full((D_HID, D_HID)), full((D_HID,)),
        ],
        out_specs=[
            pl.BlockSpec((blk, D_OUT), lambda i: (i, 0)),
            pl.BlockSpec((blk, D_OUT), lambda i: (i, 0)),
            pl.BlockSpec((2, blk, 64), lambda i: (0, i, 0)),
        ],
        out_shape=[
            jax.ShapeDtypeStruct((N_PAD, D_OUT), jnp.float32),
            jax.ShapeDtypeStruct((N_PAD, D_OUT), jnp.float32),
            jax.ShapeDtypeStruct((2, N_PAD, 64), jnp.float32),
        ],
    )(x_p, Wp1, bp1, Wp2, bp2, Wp3, bp3, parsing0, W1, b1, Wc0, bc0)


def _stats_body(ew_ref, ab_ref):
    ew = ew_ref[...]
    s1 = jnp.sum(ew)
    mean = s1 / E
    var = jnp.sum((ew - mean) ** 2) / (E - 1)
    a = jnp.sqrt(1e-4 / var)
    b = 1.0 - a * mean
    col = lax.broadcasted_iota(jnp.int32, (1, 128), 1)
    ab_ref[...] = jnp.where(col == 0, a, jnp.where(col == 1, b, 0.0))


def _stats_call(ew_valid):
    return pl.pallas_call(
        _stats_body,
        out_shape=jax.ShapeDtypeStruct((1, 128), jnp.float32),
    )(ew_valid)


def _finalize_body(degp_ref, cntp_ref, ab_ref, dis_ref, alpha_ref, beta_ref):
    a = ab_ref[0, 0]
    b = ab_ref[0, 1]
    deg = a * (degp_ref[0] + degp_ref[1]) + b * (cntp_ref[0] + cntp_ref[1])
    good = deg > 1e-6
    deg_c = jnp.maximum(deg, 1e-6)
    dis = jnp.where(good, lax.rsqrt(deg_c), 0.0)
    den = MU + jnp.where(good, 1.0, 0.0)
    alpha = 1.0 / den
    dis_ref[...] = dis
    alpha_ref[...] = alpha
    beta_ref[...] = MU * alpha


def _finalize_call(degp, cntp, ab):
    return pl.pallas_call(
        _finalize_body,
        out_shape=[
            jax.ShapeDtypeStruct((80, 128), jnp.float32),
            jax.ShapeDtypeStruct((80, 128), jnp.float32),
            jax.ShapeDtypeStruct((80, 128), jnp.float32),
        ],
    )(degp, cntp, ab)


def _combine_body(p0_ref, p1_ref, f0_ref, alpha_ref, beta_ref, out_ref):
    out_ref[...] = (alpha_ref[...] * (p0_ref[...] + p1_ref[...])
                    + beta_ref[...] * f0_ref[...])


def _combine_split_body(p_ref, f0_ref, alpha_ref, beta_ref, out_ref):
    al = alpha_ref[...][None, :, :]
    be = beta_ref[...][None, :, :]
    out_ref[...] = al * p_ref[...] + be * f0_ref[...]


def _combine_split_call(p, f0h, alpha_c, beta_c):
    blk = 640
    return pl.pallas_call(
        _combine_split_body,
        grid=(N_PAD // blk,),
        in_specs=[
            pl.BlockSpec((2, blk, 64), lambda i: (0, i, 0)),
            pl.BlockSpec((2, blk, 64), lambda i: (0, i, 0)),
            pl.BlockSpec((blk, 1), lambda i: (i, 0)),
            pl.BlockSpec((blk, 1), lambda i: (i, 0)),
        ],
        out_specs=pl.BlockSpec((2, blk, 64), lambda i: (0, i, 0)),
        out_shape=jax.ShapeDtypeStruct((2, N_PAD, 64), jnp.float32),
    )(p, f0h, alpha_c, beta_c)


def _combine_call(p0, p1, f0, alpha_c, beta_c, d):
    blk = 640
    return pl.pallas_call(
        _combine_body,
        grid=(N_PAD // blk,),
        in_specs=[
            pl.BlockSpec((blk, d), lambda i: (i, 0)),
            pl.BlockSpec((blk, d), lambda i: (i, 0)),
            pl.BlockSpec((blk, d), lambda i: (i, 0)),
            pl.BlockSpec((blk, 1), lambda i: (i, 0)),
            pl.BlockSpec((blk, 1), lambda i: (i, 0)),
        ],
        out_specs=pl.BlockSpec((blk, d), lambda i: (i, 0)),
        out_shape=jax.ShapeDtypeStruct((N_PAD, d), jnp.float32),
    )(p0, p1, f0, alpha_c, beta_c)


def _comb_mm_body(p_ref, f0_ref, alpha_ref, beta_ref, wc1, bc1, out_ref):
    al = alpha_ref[...][None, :, :]
    be = beta_ref[...][None, :, :]
    f2 = al * p_ref[...] + be * f0_ref[...]
    h2 = jnp.maximum(f2, 0.0)
    w = wc1[...]
    out_ref[...] = (_dot(h2[0], w[:64]) + _dot(h2[1], w[64:])
                    + bc1[...][None, :])


def _comb_mm_call(p, f0h, alpha_c, beta_c, Wc1, bc1):
    blk = 640
    return pl.pallas_call(
        _comb_mm_body,
        grid=(N_PAD // blk,),
        in_specs=[
            pl.BlockSpec((2, blk, 64), lambda i: (0, i, 0)),
            pl.BlockSpec((2, blk, 64), lambda i: (0, i, 0)),
            pl.BlockSpec((blk, 1), lambda i: (i, 0)),
            pl.BlockSpec((blk, 1), lambda i: (i, 0)),
            pl.BlockSpec((D_HID, D_OUT), lambda i: (0, 0)),
            pl.BlockSpec((D_OUT,), lambda i: (0,)),
        ],
        out_specs=pl.BlockSpec((blk, D_OUT), lambda i: (i, 0)),
        out_shape=jax.ShapeDtypeStruct((N_PAD, D_OUT), jnp.float32),
    )(p, f0h, alpha_c, beta_c, Wc1, bc1)


_MESH = plsc.VectorSubcoreMesh(core_axis_name="c", subcore_axis_name="s",
                               num_cores=NC, num_subcores=NS)

_SC_PARAMS = pltpu.CompilerParams()
if "needs_layout_passes" in pltpu.CompilerParams.__dataclass_fields__:
    _SC_PARAMS = dataclasses.replace(_SC_PARAMS, needs_layout_passes=False)
if "use_tc_tiling_on_sc" in pltpu.CompilerParams.__dataclass_fields__:
    _SC_PARAMS = dataclasses.replace(_SC_PARAMS, use_tc_tiling_on_sc=False)


def _iota16():
    return lax.iota(jnp.int32, LL)


def _zero_vec_ref(ref, n):
    z = jnp.zeros((LL,), jnp.float32)

    @pl.loop(0, n // LL)
    def _(i):
        ref[pl.ds(i * LL, LL)] = z


def _ew_deg_body(logits_hbm, q_hbm, row_hbm, col_hbm,
                 ew_hbm, degp_hbm, cntp_hbm,
                 ridx, cidx, abuf, bbuf, ewbuf, obuf, zbuf,
                 deg_sh, cnt_sh, gsA, gsB, wsem, dsem, csem):
    cid = lax.axis_index("c")
    sid = lax.axis_index("s")
    wid = cid * NS + sid
    base = wid * NCH

    pltpu.sync_copy(row_hbm.at[pl.ds(base, NCH)], ridx)
    pltpu.sync_copy(col_hbm.at[pl.ds(base, NCH)], cidx)

    _zero_vec_ref(zbuf, NPT)
    pltpu.sync_copy(zbuf, deg_sh.at[pl.ds(sid * NPT, NPT)])
    pltpu.sync_copy(zbuf, cnt_sh.at[pl.ds(sid * NPT, NPT)])

    ones = jnp.ones((LL,), jnp.float32)

    @pl.loop(0, ECH // LL)
    def _(i):
        obuf[pl.ds(i * LL, LL)] = ones

    plsc.subcore_barrier()

    def abuf_at(b):
        return abuf.at[pl.ds(b * ECH, ECH)]

    def bbuf_at(b):
        return bbuf.at[pl.ds(b * ECH, ECH)]

    def ewb_at(b):
        return ewbuf.at[pl.ds(b * ECH, ECH)]

    def issue_gathers(k, b):
        pltpu.async_copy(logits_hbm.at[ridx.at[k]], abuf_at(b), gsA)
        pltpu.async_copy(q_hbm.at[cidx.at[k]], bbuf_at(b), gsB)

    def wait_gathers(k, b):
        pltpu.make_async_copy(logits_hbm.at[ridx.at[k]], abuf_at(b), gsA).wait()
        pltpu.make_async_copy(q_hbm.at[cidx.at[k]], bbuf_at(b), gsB).wait()

    def issue_outs(k, b):
        pltpu.async_copy(ewb_at(b), ew_hbm.at[base + k], wsem)
        pltpu.async_copy(ewb_at(b), deg_sh.at[ridx.at[k]], dsem, add=True)
        pltpu.async_copy(obuf, cnt_sh.at[ridx.at[k]], csem, add=True)

    def wait_outs(k, b):
        pltpu.make_async_copy(ewb_at(b), ew_hbm.at[base + k], wsem).wait()
        pltpu.make_async_copy(ewb_at(b), deg_sh.at[ridx.at[k]], dsem).wait()

    issue_gathers(0, 0)

    @pl.loop(0, NCH // 2)
    def _(outer):
        k0 = outer * 2
        for i in range(2):
            k = k0 + i
            b = i
            wait_gathers(k, b)

            @pl.when(k + 1 < NCH)
            def _():
                issue_gathers(k + 1, 1 - b)

            @pl.when(k >= 2)
            def _():
                wait_outs(k - 2, b)

            ab = abuf_at(b)
            bb = bbuf_at(b)
            eb = ewb_at(b)
            for g in range(ECH // LL):
                erow = g * LL + _iota16()
                acc = jnp.zeros((LL,), jnp.float32)
                for c in range(D_OUT):
                    fcol = jnp.full((LL,), c, jnp.int32)
                    av = plsc.load_gather(ab, [erow, fcol])
                    bv = plsc.load_gather(bb, [erow, fcol])
                    acc = acc + av * bv
                eb[pl.ds(g * LL, LL)] = acc
            issue_outs(k, b)

    for k in range(NCH - 2, NCH):
        wait_outs(k, k % 2)

    @pl.loop(0, NCH)
    def _(k):
        pltpu.make_async_copy(obuf, cnt_sh.at[ridx.at[k]], csem).wait()

    plsc.subcore_barrier()
    pltpu.sync_copy(deg_sh.at[pl.ds(sid * NPT, NPT)],
                    degp_hbm.at[cid, pl.ds(sid * NPT, NPT)])
    pltpu.sync_copy(cnt_sh.at[pl.ds(sid * NPT, NPT)],
                    cntp_hbm.at[cid, pl.ds(sid * NPT, NPT)])


def _ew_deg_call(logits_p, q_p, row2d, col2d):
    kern = pl.kernel(
        _ew_deg_body,
        out_type=[
            jax.ShapeDtypeStruct((E_PAD // ECH, ECH), jnp.float32),
            jax.ShapeDtypeStruct((NC, N_PAD), jnp.float32),
            jax.ShapeDtypeStruct((NC, N_PAD), jnp.float32),
        ],
        mesh=_MESH,
        compiler_params=_SC_PARAMS,
        scratch_types=[
            pltpu.VMEM((NCH, ECH), jnp.int32),
            pltpu.VMEM((NCH, ECH), jnp.int32),
            pltpu.VMEM((2 * ECH, D_OUT), jnp.float32),
            pltpu.VMEM((2 * ECH, D_OUT), jnp.float32),
            pltpu.VMEM((2 * ECH,), jnp.float32),
            pltpu.VMEM((ECH,), jnp.float32),
            pltpu.VMEM((NPT,), jnp.float32),
            pltpu.VMEM_SHARED((N_PAD,), jnp.float32),
            pltpu.VMEM_SHARED((N_PAD,), jnp.float32),
            pltpu.SemaphoreType.DMA, pltpu.SemaphoreType.DMA,
            pltpu.SemaphoreType.DMA, pltpu.SemaphoreType.DMA,
            pltpu.SemaphoreType.DMA,
        ],
    )
    return kern(logits_p, q_p, row2d, col2d)


def _coef_body(ew_hbm, row_hbm, col_hbm, dis_hbm, a_hbm, b_hbm, coef_hbm,
               ridx, cidx, ewbuf, cfbuf, disv, avr, bvr):
    cid = lax.axis_index("c")
    sid = lax.axis_index("s")
    wid = cid * NS + sid
    base = wid * NCH

    pltpu.sync_copy(row_hbm.at[pl.ds(base, NCH)], ridx)
    pltpu.sync_copy(col_hbm.at[pl.ds(base, NCH)], cidx)
    pltpu.sync_copy(dis_hbm, disv)
    pltpu.sync_copy(a_hbm, avr)
    pltpu.sync_copy(b_hbm, bvr)
    av = avr[...]
    bv = bvr[...]

    @pl.loop(0, NCH)
    def _(j):
        pltpu.sync_copy(ew_hbm.at[base + j], ewbuf)
        for g in range(ECH // LL):
            sl = pl.ds(g * LL, LL)
            rv = ridx[j, sl]
            cv = cidx[j, sl]
            dr = plsc.load_gather(disv, [rv])
            dc = plsc.load_gather(disv, [cv])
            cfbuf[sl] = (av * ewbuf[sl] + bv) * dr * dc
        pltpu.sync_copy(cfbuf, coef_hbm.at[base + j])


def _coef_call(ew2d, row2d, col2d, dis_flat, a16, b16):
    kern = pl.kernel(
        _coef_body,
        out_type=jax.ShapeDtypeStruct((E_PAD // ECH, ECH), jnp.float32),
        mesh=_MESH,
        compiler_params=_SC_PARAMS,
        scratch_types=[
            pltpu.VMEM((NCH, ECH), jnp.int32),
            pltpu.VMEM((NCH, ECH), jnp.int32),
            pltpu.VMEM((ECH,), jnp.float32),
            pltpu.VMEM((ECH,), jnp.float32),
            pltpu.VMEM((N_PAD,), jnp.float32),
            pltpu.VMEM((LL,), jnp.float32),
            pltpu.VMEM((LL,), jnp.float32),
        ],
    )
    return kern(ew2d, row2d, col2d, dis_flat, a16, b16)


def _spmv_body(d, nb, nch, split, f_hbm, row_hbm, col_hbm, coef_hbm, aggp_hbm,
               ridx, cidx, cvm, rows, agg_sh, *sems):
    gsems = sems[:nb]
    ssems = sems[nb:]
    pd = nb // 2
    cid = lax.axis_index("c")
    sid = lax.axis_index("s")
    if split:
        base = sid * nch
        table = f_hbm.at[cid]
    else:
        base = (cid * NS + sid) * nch
        table = f_hbm

    pltpu.sync_copy(row_hbm.at[pl.ds(base, nch)], ridx)
    pltpu.sync_copy(col_hbm.at[pl.ds(base, nch)], cidx)
    pltpu.sync_copy(coef_hbm.at[pl.ds(base, nch)], cvm)

    z = jnp.zeros((LL,), jnp.float32)

    @pl.loop(0, 64)
    def _(i):
        for jj in range(d // LL):
            rows[i, pl.ds(jj * LL, LL)] = z

    zsrc = rows.at[pl.ds(0, 64)]

    @pl.loop(0, NPT // 64)
    def _(k):
        pltpu.sync_copy(zsrc, agg_sh.at[pl.ds(sid * NPT + k * 64, 64)])

    plsc.subcore_barrier()

    def rows_at(b):
        return rows.at[pl.ds(b * ECH, ECH)]

    def issue_gather(k, b):
        pltpu.async_copy(table.at[cidx.at[k]], rows_at(b), gsems[b])

    def wait_gather(k, b):
        pltpu.make_async_copy(table.at[cidx.at[k]], rows_at(b), gsems[b]).wait()

    def issue_scatter(k, b):
        pltpu.async_copy(rows_at(b), agg_sh.at[ridx.at[k]], ssems[b], add=True)

    def wait_scatter(k, b):
        pltpu.make_async_copy(rows_at(b), agg_sh.at[ridx.at[k]],
                              ssems[b]).wait()

    def scale(k, b):
        kv = jnp.full((LL,), k, jnp.int32)

        @pl.loop(0, ECH, step=4)
        def _(e):
            r = rows_at(b)
            cbs = [plsc.load_gather(
                cvm, [kv, jnp.full((LL,), e + u, jnp.int32)])
                for u in range(4)]
            for u in range(4):
                for jj in range(d // LL):
                    sl = pl.ds(jj * LL, LL)
                    r[e + u, sl] = r[e + u, sl] * cbs[u]

    for k in range(pd):
        issue_gather(k, k)

    @pl.loop(0, nch // nb)
    def _(outer):
        k0 = outer * nb
        for i in range(nb):
            k = k0 + i
            wait_gather(k, i)
            bp = (i + pd) % nb
            kp = k + pd
            kold = kp - nb

            @pl.when(kp < nch)
            def _():
                @pl.when(kold >= 0)
                def _():
                    wait_scatter(kold, bp)
                issue_gather(kp, bp)

            scale(k, i)
            issue_scatter(k, i)

    for k in range(nch - nb, nch):
        wait_scatter(k, k % nb)

    plsc.subcore_barrier()
    pltpu.sync_copy(agg_sh.at[pl.ds(sid * NPT, NPT)],
                    aggp_hbm.at[cid, pl.ds(sid * NPT, NPT)])


@functools.cache
def _spmv_kernel(d):
    split = d == 64
    nb = 4
    nch = NCH * NC if split else NCH
    sems = [pltpu.SemaphoreType.DMA] * (2 * nb)
    out_t = jax.ShapeDtypeStruct((NC, N_PAD, d), jnp.float32)
    return pl.kernel(
        functools.partial(_spmv_body, d, nb, nch, split),
        out_type=out_t,
        mesh=_MESH,
        compiler_params=_SC_PARAMS,
        scratch_types=[
            pltpu.VMEM((nch, ECH), jnp.int32),
            pltpu.VMEM((nch, ECH), jnp.int32),
            pltpu.VMEM((nch, ECH), jnp.float32),
            pltpu.VMEM((nb * ECH, d), jnp.float32),
            pltpu.VMEM_SHARED((N_PAD, d), jnp.float32),
        ] + sems,
    )


def _spmv_call(f, row2d, col2d, coef2d, d):
    return _spmv_kernel(d)(f, row2d, col2d, coef2d)


def kernel(x, edge_index, W1, b1, Wp1, bp1, Wp2, bp2, Wp3, bp3, parsing0,
           Wc0, bc0, Wc1, bc1):
    x_p = jnp.pad(x, ((0, N_PAD - N), (0, 0)))
    row = edge_index[0]
    col = edge_index[1]
    row_p = jnp.concatenate(
        [row, jnp.full((E_PAD - E,), N, jnp.int32)]).reshape(E_PAD // ECH, ECH)
    col_p = jnp.concatenate(
        [col, jnp.zeros((E_PAD - E,), jnp.int32)]).reshape(E_PAD // ECH, ECH)

    logits_p, q_p, f0h = _front_call(
        x_p, Wp1, bp1, Wp2, bp2, Wp3, bp3, parsing0, W1, b1, Wc0, bc0)

    ew2d, degp, cntp = _ew_deg_call(logits_p, q_p, row_p, col_p)

    ew_valid = ew2d.reshape(-1)[:E].reshape(E // 128, 128)
    ab = _stats_call(ew_valid)

    dis, alpha, beta = _finalize_call(
        degp.reshape(NC, N_PAD // 128, 128),
        cntp.reshape(NC, N_PAD // 128, 128), ab)

    dis_flat = dis.reshape(N_PAD)
    a16 = jnp.broadcast_to(ab.reshape(128)[0], (LL,))
    b16 = jnp.broadcast_to(ab.reshape(128)[1], (LL,))
    coef2d = _coef_call(ew2d, row_p, col_p, dis_flat, a16, b16)

    alpha_c = alpha.reshape(N_PAD, 1)
    beta_c = beta.reshape(N_PAD, 1)

    p = _spmv_call(f0h, row_p, col_p, coef2d, 64)
    f1h = _combine_split_call(p, f0h, alpha_c, beta_c)
    p = _spmv_call(f1h, row_p, col_p, coef2d, 64)
    g0 = _comb_mm_call(p, f0h, alpha_c, beta_c, Wc1, bc1)

    p = _spmv_call(g0, row_p, col_p, coef2d, D_OUT)
    g1 = _combine_call(p[0], p[1], g0, alpha_c, beta_c, D_OUT)
    p = _spmv_call(g1, row_p, col_p, coef2d, D_OUT)
    g2 = _combine_call(p[0], p[1], g0, alpha_c, beta_c, D_OUT)

    return g2[:N]


# speedup vs baseline: 8.3863x; 1.0295x over previous
import dataclasses
import functools

import jax
import jax.numpy as jnp
from jax import lax
from jax.experimental import pallas as pl
from jax.experimental.pallas import tpu as pltpu
from jax.experimental.pallas import tpu_sc as plsc

N = 10000
E = 160000
D_IN, D_HID, D_OUT = 256, 128, 16
MU = 0.1
SCALING = 2.0

NC, NS, LL = 2, 16, 16
NW = NC * NS
N_PAD = 10240
E_PAD = 163840
EPT = E_PAD // NW
ECH = 128
NCH = EPT // ECH
NPT = N_PAD // NS

_P_HIGH = lax.Precision.HIGHEST


def _dot(a, b):
    return lax.dot_general(a, b, (((1,), (0,)), ((), ())),
                           precision=_P_HIGH, preferred_element_type=jnp.float32)


def _front_body(x_ref, wp1, bp1, wp2, bp2, wp3, bp3, pars, w1, b1, wc0, bc0,
                logits_ref, q_ref, f0_ref):
    xb = x_ref[...]
    hp = jnp.maximum(_dot(xb, wp1[...]) + bp1[...][None, :], 0.0)
    hp = jnp.maximum(_dot(hp, wp2[...]) + bp2[...][None, :], 0.0)
    logits = _dot(hp, wp3[...]) + bp3[...][None, :]
    logits_ref[...] = logits
    parsing = jnp.maximum(SCALING * pars[...], 0.0)
    q_ref[...] = _dot(logits, parsing)
    h1 = _dot(xb, w1[...]) + b1[...][None, :]
    f0 = _dot(h1, wc0[...]) + bc0[...][None, :]
    f0_ref[0] = f0[:, :64]
    f0_ref[1] = f0[:, 64:]


def _front_call(x_p, Wp1, bp1, Wp2, bp2, Wp3, bp3, parsing0, W1, b1, Wc0, bc0):
    blk = 640
    grid = N_PAD // blk
    full = lambda shape: pl.BlockSpec(shape, lambda i: (0,) * len(shape))
    return pl.pallas_call(
        _front_body,
        grid=(grid,),
        in_specs=[
            pl.BlockSpec((blk, D_IN), lambda i: (i, 0)),
            full((D_IN, 512)), full((512,)),
            full((512, 64)), full((64,)),
            full((64, D_OUT)), full((D_OUT,)),
            full((D_OUT, D_OUT)),
            full((D_IN, D_HID)), full((D_HID,)),
            full((D_HID, D_HID)), full((D_HID,)),
        ],
        out_specs=[
            pl.BlockSpec((blk, D_OUT), lambda i: (i, 0)),
            pl.BlockSpec((blk, D_OUT), lambda i: (i, 0)),
            pl.BlockSpec((2, blk, 64), lambda i: (0, i, 0)),
        ],
        out_shape=[
            jax.ShapeDtypeStruct((N_PAD, D_OUT), jnp.float32),
            jax.ShapeDtypeStruct((N_PAD, D_OUT), jnp.float32),
            jax.ShapeDtypeStruct((2, N_PAD, 64), jnp.float32),
        ],
    )(x_p, Wp1, bp1, Wp2, bp2, Wp3, bp3, parsing0, W1, b1, Wc0, bc0)


def _stats_body(ew_ref, ab_ref):
    ew = ew_ref[...]
    s1 = jnp.sum(ew)
    mean = s1 / E
    var = jnp.sum((ew - mean) ** 2) / (E - 1)
    a = jnp.sqrt(1e-4 / var)
    b = 1.0 - a * mean
    col = lax.broadcasted_iota(jnp.int32, (1, 128), 1)
    ab_ref[...] = jnp.where(col == 0, a, jnp.where(col == 1, b, 0.0))


def _stats_call(ew_valid):
    return pl.pallas_call(
        _stats_body,
        out_shape=jax.ShapeDtypeStruct((1, 128), jnp.float32),
    )(ew_valid)


def _finalize_body(degp_ref, cntp_ref, ab_ref, dis_ref, alpha_ref, beta_ref):
    a = ab_ref[0, 0]
    b = ab_ref[0, 1]
    deg = a * (degp_ref[0] + degp_ref[1]) + b * (cntp_ref[0] + cntp_ref[1])
    good = deg > 1e-6
    deg_c = jnp.maximum(deg, 1e-6)
    dis = jnp.where(good, lax.rsqrt(deg_c), 0.0)
    den = MU + jnp.where(good, 1.0, 0.0)
    alpha = 1.0 / den
    dis_ref[...] = dis
    alpha_ref[...] = alpha
    beta_ref[...] = MU * alpha


def _finalize_call(degp, cntp, ab):
    return pl.pallas_call(
        _finalize_body,
        out_shape=[
            jax.ShapeDtypeStruct((80, 128), jnp.float32),
            jax.ShapeDtypeStruct((80, 128), jnp.float32),
            jax.ShapeDtypeStruct((80, 128), jnp.float32),
        ],
    )(degp, cntp, ab)


def _combine_body(p0_ref, p1_ref, f0_ref, alpha_ref, beta_ref, out_ref):
    out_ref[...] = (alpha_ref[...] * (p0_ref[...] + p1_ref[...])
                    + beta_ref[...] * f0_ref[...])


def _combine_split_body(p_ref, f0_ref, alpha_ref, beta_ref, out_ref):
    al = alpha_ref[...][None, :, :]
    be = beta_ref[...][None, :, :]
    out_ref[...] = al * p_ref[...] + be * f0_ref[...]


def _combine_split_call(p, f0h, alpha_c, beta_c):
    blk = 640
    return pl.pallas_call(
        _combine_split_body,
        grid=(N_PAD // blk,),
        in_specs=[
            pl.BlockSpec((2, blk, 64), lambda i: (0, i, 0)),
            pl.BlockSpec((2, blk, 64), lambda i: (0, i, 0)),
            pl.BlockSpec((blk, 1), lambda i: (i, 0)),
            pl.BlockSpec((blk, 1), lambda i: (i, 0)),
        ],
        out_specs=pl.BlockSpec((2, blk, 64), lambda i: (0, i, 0)),
        out_shape=jax.ShapeDtypeStruct((2, N_PAD, 64), jnp.float32),
    )(p, f0h, alpha_c, beta_c)


def _combine_call(p0, p1, f0, alpha_c, beta_c, d):
    blk = 640
    return pl.pallas_call(
        _combine_body,
        grid=(N_PAD // blk,),
        in_specs=[
            pl.BlockSpec((blk, d), lambda i: (i, 0)),
            pl.BlockSpec((blk, d), lambda i: (i, 0)),
            pl.BlockSpec((blk, d), lambda i: (i, 0)),
            pl.BlockSpec((blk, 1), lambda i: (i, 0)),
            pl.BlockSpec((blk, 1), lambda i: (i, 0)),
        ],
        out_specs=pl.BlockSpec((blk, d), lambda i: (i, 0)),
        out_shape=jax.ShapeDtypeStruct((N_PAD, d), jnp.float32),
    )(p0, p1, f0, alpha_c, beta_c)


def _comb_mm_body(p_ref, f0_ref, alpha_ref, beta_ref, wc1, bc1, out_ref):
    al = alpha_ref[...][None, :, :]
    be = beta_ref[...][None, :, :]
    f2 = al * p_ref[...] + be * f0_ref[...]
    h2 = jnp.maximum(f2, 0.0)
    w = wc1[...]
    out_ref[...] = (_dot(h2[0], w[:64]) + _dot(h2[1], w[64:])
                    + bc1[...][None, :])


def _comb_mm_call(p, f0h, alpha_c, beta_c, Wc1, bc1):
    blk = 640
    return pl.pallas_call(
        _comb_mm_body,
        grid=(N_PAD // blk,),
        in_specs=[
            pl.BlockSpec((2, blk, 64), lambda i: (0, i, 0)),
            pl.BlockSpec((2, blk, 64), lambda i: (0, i, 0)),
            pl.BlockSpec((blk, 1), lambda i: (i, 0)),
            pl.BlockSpec((blk, 1), lambda i: (i, 0)),
            pl.BlockSpec((D_HID, D_OUT), lambda i: (0, 0)),
            pl.BlockSpec((D_OUT,), lambda i: (0,)),
        ],
        out_specs=pl.BlockSpec((blk, D_OUT), lambda i: (i, 0)),
        out_shape=jax.ShapeDtypeStruct((N_PAD, D_OUT), jnp.float32),
    )(p, f0h, alpha_c, beta_c, Wc1, bc1)


_MESH = plsc.VectorSubcoreMesh(core_axis_name="c", subcore_axis_name="s",
                               num_cores=NC, num_subcores=NS)

_SC_PARAMS = pltpu.CompilerParams()
if "needs_layout_passes" in pltpu.CompilerParams.__dataclass_fields__:
    _SC_PARAMS = dataclasses.replace(_SC_PARAMS, needs_layout_passes=False)
if "use_tc_tiling_on_sc" in pltpu.CompilerParams.__dataclass_fields__:
    _SC_PARAMS = dataclasses.replace(_SC_PARAMS, use_tc_tiling_on_sc=False)


def _iota16():
    return lax.iota(jnp.int32, LL)


def _zero_vec_ref(ref, n):
    z = jnp.zeros((LL,), jnp.float32)

    @pl.loop(0, n // LL)
    def _(i):
        ref[pl.ds(i * LL, LL)] = z


def _ew_deg_body(logits_hbm, q_hbm, row_hbm, col_hbm,
                 ew_hbm, degp_hbm, cntp_hbm,
                 ridx, cidx, abuf, bbuf, ewbuf, obuf, zbuf,
                 deg_sh, cnt_sh, gsA, gsB, wsem, dsem, csem):
    cid = lax.axis_index("c")
    sid = lax.axis_index("s")
    wid = cid * NS + sid
    base = wid * NCH

    pltpu.sync_copy(row_hbm.at[pl.ds(base, NCH)], ridx)
    pltpu.sync_copy(col_hbm.at[pl.ds(base, NCH)], cidx)

    _zero_vec_ref(zbuf, NPT)
    pltpu.sync_copy(zbuf, deg_sh.at[pl.ds(sid * NPT, NPT)])
    pltpu.sync_copy(zbuf, cnt_sh.at[pl.ds(sid * NPT, NPT)])

    ones = jnp.ones((LL,), jnp.float32)

    @pl.loop(0, ECH // LL)
    def _(i):
        obuf[pl.ds(i * LL, LL)] = ones

    plsc.subcore_barrier()

    def abuf_at(b):
        return abuf.at[pl.ds(b * ECH, ECH)]

    def bbuf_at(b):
        return bbuf.at[pl.ds(b * ECH, ECH)]

    def ewb_at(b):
        return ewbuf.at[pl.ds(b * ECH, ECH)]

    def issue_gathers(k, b):
        pltpu.async_copy(logits_hbm.at[ridx.at[k]], abuf_at(b), gsA)
        pltpu.async_copy(q_hbm.at[cidx.at[k]], bbuf_at(b), gsB)

    def wait_gathers(k, b):
        pltpu.make_async_copy(logits_hbm.at[ridx.at[k]], abuf_at(b), gsA).wait()
        pltpu.make_async_copy(q_hbm.at[cidx.at[k]], bbuf_at(b), gsB).wait()

    def issue_outs(k, b):
        pltpu.async_copy(ewb_at(b), ew_hbm.at[base + k], wsem)
        pltpu.async_copy(ewb_at(b), deg_sh.at[ridx.at[k]], dsem, add=True)
        pltpu.async_copy(obuf, cnt_sh.at[ridx.at[k]], csem, add=True)

    def wait_outs(k, b):
        pltpu.make_async_copy(ewb_at(b), ew_hbm.at[base + k], wsem).wait()
        pltpu.make_async_copy(ewb_at(b), deg_sh.at[ridx.at[k]], dsem).wait()

    issue_gathers(0, 0)

    @pl.loop(0, NCH // 2)
    def _(outer):
        k0 = outer * 2
        for i in range(2):
            k = k0 + i
            b = i
            wait_gathers(k, b)

            @pl.when(k + 1 < NCH)
            def _():
                issue_gathers(k + 1, 1 - b)

            @pl.when(k >= 2)
            def _():
                wait_outs(k - 2, b)

            ab = abuf_at(b)
            bb = bbuf_at(b)
            eb = ewb_at(b)
            for g in range(ECH // LL):
                erow = g * LL + _iota16()
                acc = jnp.zeros((LL,), jnp.float32)
                for c in range(D_OUT):
                    fcol = jnp.full((LL,), c, jnp.int32)
                    av = plsc.load_gather(ab, [erow, fcol])
                    bv = plsc.load_gather(bb, [erow, fcol])
                    acc = acc + av * bv
                eb[pl.ds(g * LL, LL)] = acc
            issue_outs(k, b)

    for k in range(NCH - 2, NCH):
        wait_outs(k, k % 2)

    @pl.loop(0, NCH)
    def _(k):
        pltpu.make_async_copy(obuf, cnt_sh.at[ridx.at[k]], csem).wait()

    plsc.subcore_barrier()
    pltpu.sync_copy(deg_sh.at[pl.ds(sid * NPT, NPT)],
                    degp_hbm.at[cid, pl.ds(sid * NPT, NPT)])
    pltpu.sync_copy(cnt_sh.at[pl.ds(sid * NPT, NPT)],
                    cntp_hbm.at[cid, pl.ds(sid * NPT, NPT)])


def _ew_deg_call(logits_p, q_p, row2d, col2d):
    kern = pl.kernel(
        _ew_deg_body,
        out_type=[
            jax.ShapeDtypeStruct((E_PAD // ECH, ECH), jnp.float32),
            jax.ShapeDtypeStruct((NC, N_PAD), jnp.float32),
            jax.ShapeDtypeStruct((NC, N_PAD), jnp.float32),
        ],
        mesh=_MESH,
        compiler_params=_SC_PARAMS,
        scratch_types=[
            pltpu.VMEM((NCH, ECH), jnp.int32),
            pltpu.VMEM((NCH, ECH), jnp.int32),
            pltpu.VMEM((2 * ECH, D_OUT), jnp.float32),
            pltpu.VMEM((2 * ECH, D_OUT), jnp.float32),
            pltpu.VMEM((2 * ECH,), jnp.float32),
            pltpu.VMEM((ECH,), jnp.float32),
            pltpu.VMEM((NPT,), jnp.float32),
            pltpu.VMEM_SHARED((N_PAD,), jnp.float32),
            pltpu.VMEM_SHARED((N_PAD,), jnp.float32),
            pltpu.SemaphoreType.DMA, pltpu.SemaphoreType.DMA,
            pltpu.SemaphoreType.DMA, pltpu.SemaphoreType.DMA,
            pltpu.SemaphoreType.DMA,
        ],
    )
    return kern(logits_p, q_p, row2d, col2d)


def _coef_body(ew_hbm, row_hbm, col_hbm, dis_hbm, a_hbm, b_hbm, coef_hbm,
               ridx, cidx, ewbuf, cfbuf, disv, avr, bvr):
    cid = lax.axis_index("c")
    sid = lax.axis_index("s")
    wid = cid * NS + sid
    base = wid * NCH

    pltpu.sync_copy(row_hbm.at[pl.ds(base, NCH)], ridx)
    pltpu.sync_copy(col_hbm.at[pl.ds(base, NCH)], cidx)
    pltpu.sync_copy(dis_hbm, disv)
    pltpu.sync_copy(a_hbm, avr)
    pltpu.sync_copy(b_hbm, bvr)
    av = avr[...]
    bv = bvr[...]

    @pl.loop(0, NCH)
    def _(j):
        pltpu.sync_copy(ew_hbm.at[base + j], ewbuf)
        for g in range(ECH // LL):
            sl = pl.ds(g * LL, LL)
            rv = ridx[j, sl]
            cv = cidx[j, sl]
            dr = plsc.load_gather(disv, [rv])
            dc = plsc.load_gather(disv, [cv])
            cfbuf[sl] = (av * ewbuf[sl] + bv) * dr * dc
        pltpu.sync_copy(cfbuf, coef_hbm.at[base + j])


def _coef_call(ew2d, row2d, col2d, dis_flat, a16, b16):
    kern = pl.kernel(
        _coef_body,
        out_type=jax.ShapeDtypeStruct((E_PAD // ECH, ECH), jnp.float32),
        mesh=_MESH,
        compiler_params=_SC_PARAMS,
        scratch_types=[
            pltpu.VMEM((NCH, ECH), jnp.int32),
            pltpu.VMEM((NCH, ECH), jnp.int32),
            pltpu.VMEM((ECH,), jnp.float32),
            pltpu.VMEM((ECH,), jnp.float32),
            pltpu.VMEM((N_PAD,), jnp.float32),
            pltpu.VMEM((LL,), jnp.float32),
            pltpu.VMEM((LL,), jnp.float32),
        ],
    )
    return kern(ew2d, row2d, col2d, dis_flat, a16, b16)


def _spmv_body(d, nb, nch, split, fuse, f_hbm, row_hbm, col_hbm, coef_hbm,
               *rest):
    if fuse:
        (alpha_hbm, beta_hbm, aggp_hbm,
         ridx, cidx, cvm, rows, abblk, agg_sh, *sems) = rest
    else:
        aggp_hbm, ridx, cidx, cvm, rows, agg_sh, *sems = rest
    gsems = sems[:nb]
    ssems = sems[nb:]
    pd = nb // 2
    cid = lax.axis_index("c")
    sid = lax.axis_index("s")
    if split:
        base = sid * nch
        table = f_hbm.at[cid]
    else:
        base = (cid * NS + sid) * nch
        table = f_hbm

    pltpu.sync_copy(row_hbm.at[pl.ds(base, nch)], ridx)
    pltpu.sync_copy(col_hbm.at[pl.ds(base, nch)], cidx)
    pltpu.sync_copy(coef_hbm.at[pl.ds(base, nch)], cvm)

    z = jnp.zeros((LL,), jnp.float32)

    @pl.loop(0, 64)
    def _(i):
        for jj in range(d // LL):
            rows[i, pl.ds(jj * LL, LL)] = z

    zsrc = rows.at[pl.ds(0, 64)]

    @pl.loop(0, NPT // 64)
    def _(k):
        pltpu.sync_copy(zsrc, agg_sh.at[pl.ds(sid * NPT + k * 64, 64)])

    plsc.subcore_barrier()

    def rows_at(b):
        return rows.at[pl.ds(b * ECH, ECH)]

    def issue_gather(k, b):
        pltpu.async_copy(table.at[cidx.at[k]], rows_at(b), gsems[b])

    def wait_gather(k, b):
        pltpu.make_async_copy(table.at[cidx.at[k]], rows_at(b), gsems[b]).wait()

    def issue_scatter(k, b):
        pltpu.async_copy(rows_at(b), agg_sh.at[ridx.at[k]], ssems[b], add=True)

    def wait_scatter(k, b):
        pltpu.make_async_copy(rows_at(b), agg_sh.at[ridx.at[k]],
                              ssems[b]).wait()

    def scale(k, b):
        kv = jnp.full((LL,), k, jnp.int32)

        @pl.loop(0, ECH, step=4)
        def _(e):
            r = rows_at(b)
            cbs = [plsc.load_gather(
                cvm, [kv, jnp.full((LL,), e + u, jnp.int32)])
                for u in range(4)]
            for u in range(4):
                for jj in range(d // LL):
                    sl = pl.ds(jj * LL, LL)
                    r[e + u, sl] = r[e + u, sl] * cbs[u]

    for k in range(pd):
        issue_gather(k, k)

    @pl.loop(0, nch // nb)
    def _(outer):
        k0 = outer * nb
        for i in range(nb):
            k = k0 + i
            wait_gather(k, i)
            bp = (i + pd) % nb
            kp = k + pd
            kold = kp - nb

            @pl.when(kp < nch)
            def _():
                @pl.when(kold >= 0)
                def _():
                    wait_scatter(kold, bp)
                issue_gather(kp, bp)

            scale(k, i)
            issue_scatter(k, i)

    for k in range(nch - nb, nch):
        wait_scatter(k, k % nb)

    plsc.subcore_barrier()
    if not fuse:
        pltpu.sync_copy(agg_sh.at[pl.ds(sid * NPT, NPT)],
                        aggp_hbm.at[cid, pl.ds(sid * NPT, NPT)])
    else:
        av = abblk.at[pl.ds(0, NPT)]
        bv = abblk.at[pl.ds(NPT, NPT)]
        pltpu.sync_copy(alpha_hbm.at[pl.ds(sid * NPT, NPT)], av)
        pltpu.sync_copy(beta_hbm.at[pl.ds(sid * NPT, NPT)], bv)
        sub = 64
        fblk = rows.at[pl.ds(0, sub)]
        ablk = rows.at[pl.ds(sub, sub)]

        @pl.loop(0, NPT // sub)
        def _(sb):
            r0 = sid * NPT + sb * sub
            pltpu.sync_copy(f_hbm.at[cid, pl.ds(r0, sub)], fblk)
            pltpu.sync_copy(agg_sh.at[pl.ds(r0, sub)], ablk)

            @pl.loop(0, sub, step=4)
            def _(rr):
                for u in range(4):
                    ri = sb * sub + rr + u
                    ab = plsc.load_gather(av, [jnp.full((LL,), ri, jnp.int32)])
                    bb = plsc.load_gather(bv, [jnp.full((LL,), ri, jnp.int32)])
                    for jj in range(d // LL):
                        sl = pl.ds(jj * LL, LL)
                        fblk[rr + u, sl] = (ab * ablk[rr + u, sl]
                                            + bb * fblk[rr + u, sl])

            pltpu.sync_copy(fblk, aggp_hbm.at[cid, pl.ds(r0, sub)])


@functools.cache
def _spmv_kernel(d, fuse=False):
    split = d == 64
    nb = 4
    nch = NCH * NC if split else NCH
    sems = [pltpu.SemaphoreType.DMA] * (2 * nb)
    out_t = jax.ShapeDtypeStruct((NC, N_PAD, d), jnp.float32)
    scratch = [
        pltpu.VMEM((nch, ECH), jnp.int32),
        pltpu.VMEM((nch, ECH), jnp.int32),
        pltpu.VMEM((nch, ECH), jnp.float32),
        pltpu.VMEM((nb * ECH, d), jnp.float32),
    ]
    if fuse:
        scratch.append(pltpu.VMEM((2 * NPT,), jnp.float32))
    scratch.append(pltpu.VMEM_SHARED((N_PAD, d), jnp.float32))
    return pl.kernel(
        functools.partial(_spmv_body, d, nb, nch, split, fuse),
        out_type=out_t,
        mesh=_MESH,
        compiler_params=_SC_PARAMS,
        scratch_types=scratch + sems,
    )


def _spmv_call(f, row2d, col2d, coef2d, d):
    return _spmv_kernel(d)(f, row2d, col2d, coef2d)


def _spmv_combine_call(f, row2d, col2d, coef2d, alpha_f, beta_f):
    return _spmv_kernel(64, True)(f, row2d, col2d, coef2d, alpha_f, beta_f)


def kernel(x, edge_index, W1, b1, Wp1, bp1, Wp2, bp2, Wp3, bp3, parsing0,
           Wc0, bc0, Wc1, bc1):
    x_p = jnp.pad(x, ((0, N_PAD - N), (0, 0)))
    row = edge_index[0]
    col = edge_index[1]
    row_p = jnp.concatenate(
        [row, jnp.full((E_PAD - E,), N, jnp.int32)]).reshape(E_PAD // ECH, ECH)
    col_p = jnp.concatenate(
        [col, jnp.zeros((E_PAD - E,), jnp.int32)]).reshape(E_PAD // ECH, ECH)

    logits_p, q_p, f0h = _front_call(
        x_p, Wp1, bp1, Wp2, bp2, Wp3, bp3, parsing0, W1, b1, Wc0, bc0)

    ew2d, degp, cntp = _ew_deg_call(logits_p, q_p, row_p, col_p)

    ew_valid = ew2d.reshape(-1)[:E].reshape(E // 128, 128)
    ab = _stats_call(ew_valid)

    dis, alpha, beta = _finalize_call(
        degp.reshape(NC, N_PAD // 128, 128),
        cntp.reshape(NC, N_PAD // 128, 128), ab)

    dis_flat = dis.reshape(N_PAD)
    a16 = jnp.broadcast_to(ab.reshape(128)[0], (LL,))
    b16 = jnp.broadcast_to(ab.reshape(128)[1], (LL,))
    coef2d = _coef_call(ew2d, row_p, col_p, dis_flat, a16, b16)

    alpha_c = alpha.reshape(N_PAD, 1)
    beta_c = beta.reshape(N_PAD, 1)

    f1h = _spmv_combine_call(f0h, row_p, col_p, coef2d,
                             alpha.reshape(N_PAD), beta.reshape(N_PAD))
    p = _spmv_call(f1h, row_p, col_p, coef2d, 64)
    g0 = _comb_mm_call(p, f0h, alpha_c, beta_c, Wc1, bc1)

    p = _spmv_call(g0, row_p, col_p, coef2d, D_OUT)
    g1 = _combine_call(p[0], p[1], g0, alpha_c, beta_c, D_OUT)
    p = _spmv_call(g1, row_p, col_p, coef2d, D_OUT)
    g2 = _combine_call(p[0], p[1], g0, alpha_c, beta_c, D_OUT)

    return g2[:N]


# speedup vs baseline: 8.4910x; 1.0125x over previous
import dataclasses
import functools

import jax
import jax.numpy as jnp
from jax import lax
from jax.experimental import pallas as pl
from jax.experimental.pallas import tpu as pltpu
from jax.experimental.pallas import tpu_sc as plsc

N = 10000
E = 160000
D_IN, D_HID, D_OUT = 256, 128, 16
MU = 0.1
SCALING = 2.0

NC, NS, LL = 2, 16, 16
NW = NC * NS
N_PAD = 10240
E_PAD = 163840
EPT = E_PAD // NW
ECH = 128
NCH = EPT // ECH
NPT = N_PAD // NS

_P_HIGH = lax.Precision.HIGHEST


def _dot(a, b):
    return lax.dot_general(a, b, (((1,), (0,)), ((), ())),
                           precision=_P_HIGH, preferred_element_type=jnp.float32)


def _front_body(x_ref, wp1, bp1, wp2, bp2, wp3, bp3, pars, w1, b1, wc0, bc0,
                logits_ref, q_ref, f0_ref):
    xb = x_ref[...]
    hp = jnp.maximum(_dot(xb, wp1[...]) + bp1[...][None, :], 0.0)
    hp = jnp.maximum(_dot(hp, wp2[...]) + bp2[...][None, :], 0.0)
    logits = _dot(hp, wp3[...]) + bp3[...][None, :]
    logits_ref[...] = logits
    parsing = jnp.maximum(SCALING * pars[...], 0.0)
    q_ref[...] = _dot(logits, parsing)
    h1 = _dot(xb, w1[...]) + b1[...][None, :]
    f0 = _dot(h1, wc0[...]) + bc0[...][None, :]
    f0_ref[0] = f0[:, :64]
    f0_ref[1] = f0[:, 64:]


def _front_call(x_p, Wp1, bp1, Wp2, bp2, Wp3, bp3, parsing0, W1, b1, Wc0, bc0):
    blk = 640
    grid = N_PAD // blk
    full = lambda shape: pl.BlockSpec(shape, lambda i: (0,) * len(shape))
    return pl.pallas_call(
        _front_body,
        grid=(grid,),
        in_specs=[
            pl.BlockSpec((blk, D_IN), lambda i: (i, 0)),
            full((D_IN, 512)), full((512,)),
            full((512, 64)), full((64,)),
            full((64, D_OUT)), full((D_OUT,)),
            full((D_OUT, D_OUT)),
            full((D_IN, D_HID)), full((D_HID,)),
            full((D_HID, D_HID)), full((D_HID,)),
        ],
        out_specs=[
            pl.BlockSpec((blk, D_OUT), lambda i: (i, 0)),
            pl.BlockSpec((blk, D_OUT), lambda i: (i, 0)),
            pl.BlockSpec((2, blk, 64), lambda i: (0, i, 0)),
        ],
        out_shape=[
            jax.ShapeDtypeStruct((N_PAD, D_OUT), jnp.float32),
            jax.ShapeDtypeStruct((N_PAD, D_OUT), jnp.float32),
            jax.ShapeDtypeStruct((2, N_PAD, 64), jnp.float32),
        ],
    )(x_p, Wp1, bp1, Wp2, bp2, Wp3, bp3, parsing0, W1, b1, Wc0, bc0)


def _stats_fin_body(ew_ref, degp_ref, cntp_ref, ab_ref, dis_ref, alpha_ref,
                    beta_ref):
    ew = ew_ref[...]
    s1 = jnp.sum(ew)
    mean = s1 / E
    var = jnp.sum((ew - mean) ** 2) / (E - 1)
    a = jnp.sqrt(1e-4 / var)
    b = 1.0 - a * mean
    col = lax.broadcasted_iota(jnp.int32, (1, 128), 1)
    ab_ref[...] = jnp.where(col == 0, a, jnp.where(col == 1, b, 0.0))
    deg = a * (degp_ref[0] + degp_ref[1]) + b * (cntp_ref[0] + cntp_ref[1])
    good = deg > 1e-6
    deg_c = jnp.maximum(deg, 1e-6)
    dis = jnp.where(good, lax.rsqrt(deg_c), 0.0)
    den = MU + jnp.where(good, 1.0, 0.0)
    alpha = 1.0 / den
    dis_ref[...] = dis
    alpha_ref[...] = alpha
    beta_ref[...] = MU * alpha


def _stats_fin_call(ew_valid, degp, cntp):
    return pl.pallas_call(
        _stats_fin_body,
        out_shape=[
            jax.ShapeDtypeStruct((1, 128), jnp.float32),
            jax.ShapeDtypeStruct((80, 128), jnp.float32),
            jax.ShapeDtypeStruct((80, 128), jnp.float32),
            jax.ShapeDtypeStruct((80, 128), jnp.float32),
        ],
    )(ew_valid, degp, cntp)


def _combine_body(p0_ref, p1_ref, f0_ref, alpha_ref, beta_ref, out_ref):
    out_ref[...] = (alpha_ref[...] * (p0_ref[...] + p1_ref[...])
                    + beta_ref[...] * f0_ref[...])


def _combine_split_body(p_ref, f0_ref, alpha_ref, beta_ref, out_ref):
    al = alpha_ref[...][None, :, :]
    be = beta_ref[...][None, :, :]
    out_ref[...] = al * p_ref[...] + be * f0_ref[...]


def _combine_split_call(p, f0h, alpha_c, beta_c):
    blk = 640
    return pl.pallas_call(
        _combine_split_body,
        grid=(N_PAD // blk,),
        in_specs=[
            pl.BlockSpec((2, blk, 64), lambda i: (0, i, 0)),
            pl.BlockSpec((2, blk, 64), lambda i: (0, i, 0)),
            pl.BlockSpec((blk, 1), lambda i: (i, 0)),
            pl.BlockSpec((blk, 1), lambda i: (i, 0)),
        ],
        out_specs=pl.BlockSpec((2, blk, 64), lambda i: (0, i, 0)),
        out_shape=jax.ShapeDtypeStruct((2, N_PAD, 64), jnp.float32),
    )(p, f0h, alpha_c, beta_c)


def _combine_call(p0, p1, f0, alpha_c, beta_c, d):
    blk = 640
    return pl.pallas_call(
        _combine_body,
        grid=(N_PAD // blk,),
        in_specs=[
            pl.BlockSpec((blk, d), lambda i: (i, 0)),
            pl.BlockSpec((blk, d), lambda i: (i, 0)),
            pl.BlockSpec((blk, d), lambda i: (i, 0)),
            pl.BlockSpec((blk, 1), lambda i: (i, 0)),
            pl.BlockSpec((blk, 1), lambda i: (i, 0)),
        ],
        out_specs=pl.BlockSpec((blk, d), lambda i: (i, 0)),
        out_shape=jax.ShapeDtypeStruct((N_PAD, d), jnp.float32),
    )(p0, p1, f0, alpha_c, beta_c)


def _comb_mm_body(p_ref, f0_ref, alpha_ref, beta_ref, wc1, bc1, out_ref):
    al = alpha_ref[...][None, :, :]
    be = beta_ref[...][None, :, :]
    f2 = al * p_ref[...] + be * f0_ref[...]
    h2 = jnp.maximum(f2, 0.0)
    w = wc1[...]
    out_ref[...] = (_dot(h2[0], w[:64]) + _dot(h2[1], w[64:])
                    + bc1[...][None, :])


def _comb_mm_call(p, f0h, alpha_c, beta_c, Wc1, bc1):
    blk = 640
    return pl.pallas_call(
        _comb_mm_body,
        grid=(N_PAD // blk,),
        in_specs=[
            pl.BlockSpec((2, blk, 64), lambda i: (0, i, 0)),
            pl.BlockSpec((2, blk, 64), lambda i: (0, i, 0)),
            pl.BlockSpec((blk, 1), lambda i: (i, 0)),
            pl.BlockSpec((blk, 1), lambda i: (i, 0)),
            pl.BlockSpec((D_HID, D_OUT), lambda i: (0, 0)),
            pl.BlockSpec((D_OUT,), lambda i: (0,)),
        ],
        out_specs=pl.BlockSpec((blk, D_OUT), lambda i: (i, 0)),
        out_shape=jax.ShapeDtypeStruct((N_PAD, D_OUT), jnp.float32),
    )(p, f0h, alpha_c, beta_c, Wc1, bc1)


_MESH = plsc.VectorSubcoreMesh(core_axis_name="c", subcore_axis_name="s",
                               num_cores=NC, num_subcores=NS)

_SC_PARAMS = pltpu.CompilerParams()
if "needs_layout_passes" in pltpu.CompilerParams.__dataclass_fields__:
    _SC_PARAMS = dataclasses.replace(_SC_PARAMS, needs_layout_passes=False)
if "use_tc_tiling_on_sc" in pltpu.CompilerParams.__dataclass_fields__:
    _SC_PARAMS = dataclasses.replace(_SC_PARAMS, use_tc_tiling_on_sc=False)


def _iota16():
    return lax.iota(jnp.int32, LL)


def _zero_vec_ref(ref, n):
    z = jnp.zeros((LL,), jnp.float32)

    @pl.loop(0, n // LL)
    def _(i):
        ref[pl.ds(i * LL, LL)] = z


def _ew_deg_body(logits_hbm, q_hbm, row_hbm, col_hbm,
                 ew_hbm, degp_hbm, cntp_hbm,
                 ridx, cidx, abuf, bbuf, ewbuf, obuf, zbuf,
                 deg_sh, cnt_sh, gsA, gsB, wsem, dsem, csem):
    cid = lax.axis_index("c")
    sid = lax.axis_index("s")
    wid = cid * NS + sid
    base = wid * NCH

    pltpu.sync_copy(row_hbm.at[pl.ds(base, NCH)], ridx)
    pltpu.sync_copy(col_hbm.at[pl.ds(base, NCH)], cidx)

    _zero_vec_ref(zbuf, NPT)
    pltpu.sync_copy(zbuf, deg_sh.at[pl.ds(sid * NPT, NPT)])
    pltpu.sync_copy(zbuf, cnt_sh.at[pl.ds(sid * NPT, NPT)])

    ones = jnp.ones((LL,), jnp.float32)

    @pl.loop(0, ECH // LL)
    def _(i):
        obuf[pl.ds(i * LL, LL)] = ones

    plsc.subcore_barrier()

    def abuf_at(b):
        return abuf.at[pl.ds(b * ECH, ECH)]

    def bbuf_at(b):
        return bbuf.at[pl.ds(b * ECH, ECH)]

    def ewb_at(b):
        return ewbuf.at[pl.ds(b * ECH, ECH)]

    def issue_gathers(k, b):
        pltpu.async_copy(logits_hbm.at[ridx.at[k]], abuf_at(b), gsA)
        pltpu.async_copy(q_hbm.at[cidx.at[k]], bbuf_at(b), gsB)

    def wait_gathers(k, b):
        pltpu.make_async_copy(logits_hbm.at[ridx.at[k]], abuf_at(b), gsA).wait()
        pltpu.make_async_copy(q_hbm.at[cidx.at[k]], bbuf_at(b), gsB).wait()

    def issue_outs(k, b):
        pltpu.async_copy(ewb_at(b), ew_hbm.at[base + k], wsem)
        pltpu.async_copy(ewb_at(b), deg_sh.at[ridx.at[k]], dsem, add=True)
        pltpu.async_copy(obuf, cnt_sh.at[ridx.at[k]], csem, add=True)

    def wait_outs(k, b):
        pltpu.make_async_copy(ewb_at(b), ew_hbm.at[base + k], wsem).wait()
        pltpu.make_async_copy(ewb_at(b), deg_sh.at[ridx.at[k]], dsem).wait()

    issue_gathers(0, 0)

    @pl.loop(0, NCH // 2)
    def _(outer):
        k0 = outer * 2
        for i in range(2):
            k = k0 + i
            b = i
            wait_gathers(k, b)

            @pl.when(k + 1 < NCH)
            def _():
                issue_gathers(k + 1, 1 - b)

            @pl.when(k >= 2)
            def _():
                wait_outs(k - 2, b)

            ab = abuf_at(b)
            bb = bbuf_at(b)
            eb = ewb_at(b)
            for g in range(ECH // LL):
                erow = g * LL + _iota16()
                acc = jnp.zeros((LL,), jnp.float32)
                for c in range(D_OUT):
                    fcol = jnp.full((LL,), c, jnp.int32)
                    av = plsc.load_gather(ab, [erow, fcol])
                    bv = plsc.load_gather(bb, [erow, fcol])
                    acc = acc + av * bv
                eb[pl.ds(g * LL, LL)] = acc
            issue_outs(k, b)

    for k in range(NCH - 2, NCH):
        wait_outs(k, k % 2)

    @pl.loop(0, NCH)
    def _(k):
        pltpu.make_async_copy(obuf, cnt_sh.at[ridx.at[k]], csem).wait()

    plsc.subcore_barrier()
    pltpu.sync_copy(deg_sh.at[pl.ds(sid * NPT, NPT)],
                    degp_hbm.at[cid, pl.ds(sid * NPT, NPT)])
    pltpu.sync_copy(cnt_sh.at[pl.ds(sid * NPT, NPT)],
                    cntp_hbm.at[cid, pl.ds(sid * NPT, NPT)])


def _ew_deg_call(logits_p, q_p, row2d, col2d):
    kern = pl.kernel(
        _ew_deg_body,
        out_type=[
            jax.ShapeDtypeStruct((E_PAD // ECH, ECH), jnp.float32),
            jax.ShapeDtypeStruct((NC, N_PAD), jnp.float32),
            jax.ShapeDtypeStruct((NC, N_PAD), jnp.float32),
        ],
        mesh=_MESH,
        compiler_params=_SC_PARAMS,
        scratch_types=[
            pltpu.VMEM((NCH, ECH), jnp.int32),
            pltpu.VMEM((NCH, ECH), jnp.int32),
            pltpu.VMEM((2 * ECH, D_OUT), jnp.float32),
            pltpu.VMEM((2 * ECH, D_OUT), jnp.float32),
            pltpu.VMEM((2 * ECH,), jnp.float32),
            pltpu.VMEM((ECH,), jnp.float32),
            pltpu.VMEM((NPT,), jnp.float32),
            pltpu.VMEM_SHARED((N_PAD,), jnp.float32),
            pltpu.VMEM_SHARED((N_PAD,), jnp.float32),
            pltpu.SemaphoreType.DMA, pltpu.SemaphoreType.DMA,
            pltpu.SemaphoreType.DMA, pltpu.SemaphoreType.DMA,
            pltpu.SemaphoreType.DMA,
        ],
    )
    return kern(logits_p, q_p, row2d, col2d)


def _coef_body(ew_hbm, row_hbm, col_hbm, dis_hbm, a_hbm, b_hbm, coef_hbm,
               ridx, cidx, ewbuf, cfbuf, disv, avr, bvr):
    cid = lax.axis_index("c")
    sid = lax.axis_index("s")
    wid = cid * NS + sid
    base = wid * NCH

    pltpu.sync_copy(row_hbm.at[pl.ds(base, NCH)], ridx)
    pltpu.sync_copy(col_hbm.at[pl.ds(base, NCH)], cidx)
    pltpu.sync_copy(dis_hbm, disv)
    pltpu.sync_copy(ew_hbm.at[pl.ds(base, NCH)], ewbuf)
    pltpu.sync_copy(a_hbm, avr)
    pltpu.sync_copy(b_hbm, bvr)
    av = avr[...]
    bv = bvr[...]

    @pl.loop(0, NCH)
    def _(j):
        for g in range(ECH // LL):
            sl = pl.ds(g * LL, LL)
            rv = ridx[j, sl]
            cv = cidx[j, sl]
            dr = plsc.load_gather(disv, [rv])
            dc = plsc.load_gather(disv, [cv])
            cfbuf[j, sl] = (av * ewbuf[j, sl] + bv) * dr * dc

    pltpu.sync_copy(cfbuf, coef_hbm.at[pl.ds(base, NCH)])


def _coef_call(ew2d, row2d, col2d, dis_flat, a16, b16):
    kern = pl.kernel(
        _coef_body,
        out_type=jax.ShapeDtypeStruct((E_PAD // ECH, ECH), jnp.float32),
        mesh=_MESH,
        compiler_params=_SC_PARAMS,
        scratch_types=[
            pltpu.VMEM((NCH, ECH), jnp.int32),
            pltpu.VMEM((NCH, ECH), jnp.int32),
            pltpu.VMEM((NCH, ECH), jnp.float32),
            pltpu.VMEM((NCH, ECH), jnp.float32),
            pltpu.VMEM((N_PAD,), jnp.float32),
            pltpu.VMEM((LL,), jnp.float32),
            pltpu.VMEM((LL,), jnp.float32),
        ],
    )
    return kern(ew2d, row2d, col2d, dis_flat, a16, b16)


def _spmv_body(d, nb, nch, split, fuse, f_hbm, row_hbm, col_hbm, coef_hbm,
               *rest):
    if fuse:
        (alpha_hbm, beta_hbm, aggp_hbm,
         ridx, cidx, cvm, rows, abblk, agg_sh, *sems) = rest
    else:
        aggp_hbm, ridx, cidx, cvm, rows, agg_sh, *sems = rest
    gsems = sems[:nb]
    ssems = sems[nb:]
    pd = nb // 2
    cid = lax.axis_index("c")
    sid = lax.axis_index("s")
    if split:
        base = sid * nch
        table = f_hbm.at[cid]
    else:
        base = (cid * NS + sid) * nch
        table = f_hbm

    pltpu.sync_copy(row_hbm.at[pl.ds(base, nch)], ridx)
    pltpu.sync_copy(col_hbm.at[pl.ds(base, nch)], cidx)
    pltpu.sync_copy(coef_hbm.at[pl.ds(base, nch)], cvm)

    z = jnp.zeros((LL,), jnp.float32)

    @pl.loop(0, 64)
    def _(i):
        for jj in range(d // LL):
            rows[i, pl.ds(jj * LL, LL)] = z

    zsrc = rows.at[pl.ds(0, 64)]

    @pl.loop(0, NPT // 64)
    def _(k):
        pltpu.sync_copy(zsrc, agg_sh.at[pl.ds(sid * NPT + k * 64, 64)])

    plsc.subcore_barrier()

    def rows_at(b):
        return rows.at[pl.ds(b * ECH, ECH)]

    def issue_gather(k, b):
        pltpu.async_copy(table.at[cidx.at[k]], rows_at(b), gsems[b])

    def wait_gather(k, b):
        pltpu.make_async_copy(table.at[cidx.at[k]], rows_at(b), gsems[b]).wait()

    def issue_scatter(k, b):
        pltpu.async_copy(rows_at(b), agg_sh.at[ridx.at[k]], ssems[b], add=True)

    def wait_scatter(k, b):
        pltpu.make_async_copy(rows_at(b), agg_sh.at[ridx.at[k]],
                              ssems[b]).wait()

    def scale(k, b):
        kv = jnp.full((LL,), k, jnp.int32)

        @pl.loop(0, ECH, step=4)
        def _(e):
            r = rows_at(b)
            cbs = [plsc.load_gather(
                cvm, [kv, jnp.full((LL,), e + u, jnp.int32)])
                for u in range(4)]
            for u in range(4):
                for jj in range(d // LL):
                    sl = pl.ds(jj * LL, LL)
                    r[e + u, sl] = r[e + u, sl] * cbs[u]

    for k in range(pd):
        issue_gather(k, k)

    @pl.loop(0, nch // nb)
    def _(outer):
        k0 = outer * nb
        for i in range(nb):
            k = k0 + i
            wait_gather(k, i)
            bp = (i + pd) % nb
            kp = k + pd
            kold = kp - nb

            @pl.when(kp < nch)
            def _():
                @pl.when(kold >= 0)
                def _():
                    wait_scatter(kold, bp)
                issue_gather(kp, bp)

            scale(k, i)
            issue_scatter(k, i)

    for k in range(nch - nb, nch):
        wait_scatter(k, k % nb)

    plsc.subcore_barrier()
    if not fuse:
        pltpu.sync_copy(agg_sh.at[pl.ds(sid * NPT, NPT)],
                        aggp_hbm.at[cid, pl.ds(sid * NPT, NPT)])
    else:
        av = abblk.at[pl.ds(0, NPT)]
        bv = abblk.at[pl.ds(NPT, NPT)]
        pltpu.sync_copy(alpha_hbm.at[pl.ds(sid * NPT, NPT)], av)
        pltpu.sync_copy(beta_hbm.at[pl.ds(sid * NPT, NPT)], bv)
        sub = 64
        fblk = rows.at[pl.ds(0, sub)]
        ablk = rows.at[pl.ds(sub, sub)]

        @pl.loop(0, NPT // sub)
        def _(sb):
            r0 = sid * NPT + sb * sub
            pltpu.sync_copy(f_hbm.at[cid, pl.ds(r0, sub)], fblk)
            pltpu.sync_copy(agg_sh.at[pl.ds(r0, sub)], ablk)

            @pl.loop(0, sub, step=4)
            def _(rr):
                for u in range(4):
                    ri = sb * sub + rr + u
                    ab = plsc.load_gather(av, [jnp.full((LL,), ri, jnp.int32)])
                    bb = plsc.load_gather(bv, [jnp.full((LL,), ri, jnp.int32)])
                    for jj in range(d // LL):
                        sl = pl.ds(jj * LL, LL)
                        fblk[rr + u, sl] = (ab * ablk[rr + u, sl]
                                            + bb * fblk[rr + u, sl])

            pltpu.sync_copy(fblk, aggp_hbm.at[cid, pl.ds(r0, sub)])


@functools.cache
def _spmv_kernel(d, fuse=False):
    split = d == 64
    nb = 4
    nch = NCH * NC if split else NCH
    sems = [pltpu.SemaphoreType.DMA] * (2 * nb)
    out_t = jax.ShapeDtypeStruct((NC, N_PAD, d), jnp.float32)
    scratch = [
        pltpu.VMEM((nch, ECH), jnp.int32),
        pltpu.VMEM((nch, ECH), jnp.int32),
        pltpu.VMEM((nch, ECH), jnp.float32),
        pltpu.VMEM((nb * ECH, d), jnp.float32),
    ]
    if fuse:
        scratch.append(pltpu.VMEM((2 * NPT,), jnp.float32))
    scratch.append(pltpu.VMEM_SHARED((N_PAD, d), jnp.float32))
    return pl.kernel(
        functools.partial(_spmv_body, d, nb, nch, split, fuse),
        out_type=out_t,
        mesh=_MESH,
        compiler_params=_SC_PARAMS,
        scratch_types=scratch + sems,
    )


def _spmv_call(f, row2d, col2d, coef2d, d):
    return _spmv_kernel(d)(f, row2d, col2d, coef2d)


def _spmv_combine_call(f, row2d, col2d, coef2d, alpha_f, beta_f):
    return _spmv_kernel(64, True)(f, row2d, col2d, coef2d, alpha_f, beta_f)


def kernel(x, edge_index, W1, b1, Wp1, bp1, Wp2, bp2, Wp3, bp3, parsing0,
           Wc0, bc0, Wc1, bc1):
    x_p = jnp.pad(x, ((0, N_PAD - N), (0, 0)))
    row = edge_index[0]
    col = edge_index[1]
    row_p = jnp.concatenate(
        [row, jnp.full((E_PAD - E,), N, jnp.int32)]).reshape(E_PAD // ECH, ECH)
    col_p = jnp.concatenate(
        [col, jnp.zeros((E_PAD - E,), jnp.int32)]).reshape(E_PAD // ECH, ECH)

    logits_p, q_p, f0h = _front_call(
        x_p, Wp1, bp1, Wp2, bp2, Wp3, bp3, parsing0, W1, b1, Wc0, bc0)

    ew2d, degp, cntp = _ew_deg_call(logits_p, q_p, row_p, col_p)

    ew_valid = ew2d.reshape(-1)[:E].reshape(E // 128, 128)
    ab, dis, alpha, beta = _stats_fin_call(
        ew_valid,
        degp.reshape(NC, N_PAD // 128, 128),
        cntp.reshape(NC, N_PAD // 128, 128))

    dis_flat = dis.reshape(N_PAD)
    a16 = jnp.broadcast_to(ab.reshape(128)[0], (LL,))
    b16 = jnp.broadcast_to(ab.reshape(128)[1], (LL,))
    coef2d = _coef_call(ew2d, row_p, col_p, dis_flat, a16, b16)

    alpha_c = alpha.reshape(N_PAD, 1)
    beta_c = beta.reshape(N_PAD, 1)

    f1h = _spmv_combine_call(f0h, row_p, col_p, coef2d,
                             alpha.reshape(N_PAD), beta.reshape(N_PAD))
    p = _spmv_call(f1h, row_p, col_p, coef2d, 64)
    g0 = _comb_mm_call(p, f0h, alpha_c, beta_c, Wc1, bc1)

    p = _spmv_call(g0, row_p, col_p, coef2d, D_OUT)
    g1 = _combine_call(p[0], p[1], g0, alpha_c, beta_c, D_OUT)
    p = _spmv_call(g1, row_p, col_p, coef2d, D_OUT)
    g2 = _combine_call(p[0], p[1], g0, alpha_c, beta_c, D_OUT)

    return g2[:N]


# speedup vs baseline: 8.4911x; 1.0000x over previous
import dataclasses
import functools

import jax
import jax.numpy as jnp
from jax import lax
from jax.experimental import pallas as pl
from jax.experimental.pallas import tpu as pltpu
from jax.experimental.pallas import tpu_sc as plsc

N = 10000
E = 160000
D_IN, D_HID, D_OUT = 256, 128, 16
MU = 0.1
SCALING = 2.0

NC, NS, LL = 2, 16, 16
NW = NC * NS
N_PAD = 10240
E_PAD = 163840
EPT = E_PAD // NW
ECH = 128
NCH = EPT // ECH
NPT = N_PAD // NS

_P_HIGH = lax.Precision.HIGHEST


def _dot(a, b):
    return lax.dot_general(a, b, (((1,), (0,)), ((), ())),
                           precision=_P_HIGH, preferred_element_type=jnp.float32)


def _front_body(x_ref, wp1, bp1, wp2, bp2, wp3, bp3, pars, w1, b1, wc0, bc0,
                logits_ref, q_ref, f0_ref):
    xb = x_ref[...]
    hp = jnp.maximum(_dot(xb, wp1[...]) + bp1[...][None, :], 0.0)
    hp = jnp.maximum(_dot(hp, wp2[...]) + bp2[...][None, :], 0.0)
    logits = _dot(hp, wp3[...]) + bp3[...][None, :]
    logits_ref[...] = logits
    parsing = jnp.maximum(SCALING * pars[...], 0.0)
    q_ref[...] = _dot(logits, parsing)
    h1 = _dot(xb, w1[...]) + b1[...][None, :]
    f0 = _dot(h1, wc0[...]) + bc0[...][None, :]
    f0_ref[0] = f0[:, :64]
    f0_ref[1] = f0[:, 64:]


def _front_call(x_p, Wp1, bp1, Wp2, bp2, Wp3, bp3, parsing0, W1, b1, Wc0, bc0):
    blk = 640
    grid = N_PAD // blk
    full = lambda shape: pl.BlockSpec(shape, lambda i: (0,) * len(shape))
    return pl.pallas_call(
        _front_body,
        grid=(grid,),
        in_specs=[
            pl.BlockSpec((blk, D_IN), lambda i: (i, 0)),
            full((D_IN, 512)), full((512,)),
            full((512, 64)), full((64,)),
            full((64, D_OUT)), full((D_OUT,)),
            full((D_OUT, D_OUT)),
            full((D_IN, D_HID)), full((D_HID,)),
            full((D_HID, D_HID)), full((D_HID,)),
        ],
        out_specs=[
            pl.BlockSpec((blk, D_OUT), lambda i: (i, 0)),
            pl.BlockSpec((blk, D_OUT), lambda i: (i, 0)),
            pl.BlockSpec((2, blk, 64), lambda i: (0, i, 0)),
        ],
        out_shape=[
            jax.ShapeDtypeStruct((N_PAD, D_OUT), jnp.float32),
            jax.ShapeDtypeStruct((N_PAD, D_OUT), jnp.float32),
            jax.ShapeDtypeStruct((2, N_PAD, 64), jnp.float32),
        ],
    )(x_p, Wp1, bp1, Wp2, bp2, Wp3, bp3, parsing0, W1, b1, Wc0, bc0)


def _stats_fin_body(ew_ref, degp_ref, cntp_ref, ab_ref, dis_ref, alpha_ref,
                    beta_ref):
    ew = ew_ref[...]
    s1 = jnp.sum(ew)
    mean = s1 / E
    var = jnp.sum((ew - mean) ** 2) / (E - 1)
    a = jnp.sqrt(1e-4 / var)
    b = 1.0 - a * mean
    col = lax.broadcasted_iota(jnp.int32, (1, 128), 1)
    ab_ref[...] = jnp.where(col == 0, a, jnp.where(col == 1, b, 0.0))
    deg = a * (degp_ref[0] + degp_ref[1]) + b * (cntp_ref[0] + cntp_ref[1])
    good = deg > 1e-6
    deg_c = jnp.maximum(deg, 1e-6)
    dis = jnp.where(good, lax.rsqrt(deg_c), 0.0)
    den = MU + jnp.where(good, 1.0, 0.0)
    alpha = 1.0 / den
    dis_ref[...] = dis
    alpha_ref[...] = alpha
    beta_ref[...] = MU * alpha


def _stats_fin_call(ew_valid, degp, cntp):
    return pl.pallas_call(
        _stats_fin_body,
        out_shape=[
            jax.ShapeDtypeStruct((1, 128), jnp.float32),
            jax.ShapeDtypeStruct((80, 128), jnp.float32),
            jax.ShapeDtypeStruct((80, 128), jnp.float32),
            jax.ShapeDtypeStruct((80, 128), jnp.float32),
        ],
    )(ew_valid, degp, cntp)


def _combine_body(p0_ref, p1_ref, f0_ref, alpha_ref, beta_ref, out_ref):
    out_ref[...] = (alpha_ref[...] * (p0_ref[...] + p1_ref[...])
                    + beta_ref[...] * f0_ref[...])


def _combine_call(p0, p1, f0, alpha_c, beta_c, d):
    blk = 640
    return pl.pallas_call(
        _combine_body,
        grid=(N_PAD // blk,),
        in_specs=[
            pl.BlockSpec((blk, d), lambda i: (i, 0)),
            pl.BlockSpec((blk, d), lambda i: (i, 0)),
            pl.BlockSpec((blk, d), lambda i: (i, 0)),
            pl.BlockSpec((blk, 1), lambda i: (i, 0)),
            pl.BlockSpec((blk, 1), lambda i: (i, 0)),
        ],
        out_specs=pl.BlockSpec((blk, d), lambda i: (i, 0)),
        out_shape=jax.ShapeDtypeStruct((N_PAD, d), jnp.float32),
    )(p0, p1, f0, alpha_c, beta_c)


def _comb_mm_body(p_ref, f0_ref, alpha_ref, beta_ref, wc1, bc1, out_ref):
    al = alpha_ref[...][None, :, :]
    be = beta_ref[...][None, :, :]
    f2 = al * p_ref[...] + be * f0_ref[...]
    h2 = jnp.maximum(f2, 0.0)
    w = wc1[...]
    out_ref[...] = (_dot(h2[0], w[:64]) + _dot(h2[1], w[64:])
                    + bc1[...][None, :])


def _comb_mm_call(p, f0h, alpha_c, beta_c, Wc1, bc1):
    blk = 640
    return pl.pallas_call(
        _comb_mm_body,
        grid=(N_PAD // blk,),
        in_specs=[
            pl.BlockSpec((2, blk, 64), lambda i: (0, i, 0)),
            pl.BlockSpec((2, blk, 64), lambda i: (0, i, 0)),
            pl.BlockSpec((blk, 1), lambda i: (i, 0)),
            pl.BlockSpec((blk, 1), lambda i: (i, 0)),
            pl.BlockSpec((D_HID, D_OUT), lambda i: (0, 0)),
            pl.BlockSpec((D_OUT,), lambda i: (0,)),
        ],
        out_specs=pl.BlockSpec((blk, D_OUT), lambda i: (i, 0)),
        out_shape=jax.ShapeDtypeStruct((N_PAD, D_OUT), jnp.float32),
    )(p, f0h, alpha_c, beta_c, Wc1, bc1)


_MESH = plsc.VectorSubcoreMesh(core_axis_name="c", subcore_axis_name="s",
                               num_cores=NC, num_subcores=NS)

_SC_PARAMS = pltpu.CompilerParams()
if "needs_layout_passes" in pltpu.CompilerParams.__dataclass_fields__:
    _SC_PARAMS = dataclasses.replace(_SC_PARAMS, needs_layout_passes=False)
if "use_tc_tiling_on_sc" in pltpu.CompilerParams.__dataclass_fields__:
    _SC_PARAMS = dataclasses.replace(_SC_PARAMS, use_tc_tiling_on_sc=False)


def _iota16():
    return lax.iota(jnp.int32, LL)


def _zero_vec_ref(ref, n):
    z = jnp.zeros((LL,), jnp.float32)

    @pl.loop(0, n // LL)
    def _(i):
        ref[pl.ds(i * LL, LL)] = z


def _ew_deg_body(logits_hbm, q_hbm, row_hbm, col_hbm,
                 ew_hbm, degp_hbm, cntp_hbm,
                 ridx, cidx, abuf, bbuf, ewbuf, obuf, zbuf,
                 deg_sh, cnt_sh, gsA, gsB, wsem, dsem, csem):
    cid = lax.axis_index("c")
    sid = lax.axis_index("s")
    wid = cid * NS + sid
    base = wid * NCH

    pltpu.sync_copy(row_hbm.at[pl.ds(base, NCH)], ridx)
    pltpu.sync_copy(col_hbm.at[pl.ds(base, NCH)], cidx)

    _zero_vec_ref(zbuf, NPT)
    pltpu.sync_copy(zbuf, deg_sh.at[pl.ds(sid * NPT, NPT)])
    pltpu.sync_copy(zbuf, cnt_sh.at[pl.ds(sid * NPT, NPT)])

    ones = jnp.ones((LL,), jnp.float32)

    @pl.loop(0, ECH // LL)
    def _(i):
        obuf[pl.ds(i * LL, LL)] = ones

    plsc.subcore_barrier()

    def abuf_at(b):
        return abuf.at[pl.ds(b * ECH, ECH)]

    def bbuf_at(b):
        return bbuf.at[pl.ds(b * ECH, ECH)]

    def ewb_at(b):
        return ewbuf.at[pl.ds(b * ECH, ECH)]

    def issue_gathers(k, b):
        pltpu.async_copy(logits_hbm.at[ridx.at[k]], abuf_at(b), gsA)
        pltpu.async_copy(q_hbm.at[cidx.at[k]], bbuf_at(b), gsB)

    def wait_gathers(k, b):
        pltpu.make_async_copy(logits_hbm.at[ridx.at[k]], abuf_at(b), gsA).wait()
        pltpu.make_async_copy(q_hbm.at[cidx.at[k]], bbuf_at(b), gsB).wait()

    def issue_outs(k, b):
        pltpu.async_copy(ewb_at(b), ew_hbm.at[base + k], wsem)
        pltpu.async_copy(ewb_at(b), deg_sh.at[ridx.at[k]], dsem, add=True)
        pltpu.async_copy(obuf, cnt_sh.at[ridx.at[k]], csem, add=True)

    def wait_outs(k, b):
        pltpu.make_async_copy(ewb_at(b), ew_hbm.at[base + k], wsem).wait()
        pltpu.make_async_copy(ewb_at(b), deg_sh.at[ridx.at[k]], dsem).wait()

    issue_gathers(0, 0)

    @pl.loop(0, NCH // 2)
    def _(outer):
        k0 = outer * 2
        for i in range(2):
            k = k0 + i
            b = i
            wait_gathers(k, b)

            @pl.when(k + 1 < NCH)
            def _():
                issue_gathers(k + 1, 1 - b)

            @pl.when(k >= 2)
            def _():
                wait_outs(k - 2, b)

            ab = abuf_at(b)
            bb = bbuf_at(b)
            eb = ewb_at(b)
            for g in range(ECH // LL):
                erow = g * LL + _iota16()
                acc = jnp.zeros((LL,), jnp.float32)
                for c in range(D_OUT):
                    fcol = jnp.full((LL,), c, jnp.int32)
                    av = plsc.load_gather(ab, [erow, fcol])
                    bv = plsc.load_gather(bb, [erow, fcol])
                    acc = acc + av * bv
                eb[pl.ds(g * LL, LL)] = acc
            issue_outs(k, b)

    for k in range(NCH - 2, NCH):
        wait_outs(k, k % 2)

    @pl.loop(0, NCH)
    def _(k):
        pltpu.make_async_copy(obuf, cnt_sh.at[ridx.at[k]], csem).wait()

    plsc.subcore_barrier()
    pltpu.sync_copy(deg_sh.at[pl.ds(sid * NPT, NPT)],
                    degp_hbm.at[cid, pl.ds(sid * NPT, NPT)])
    pltpu.sync_copy(cnt_sh.at[pl.ds(sid * NPT, NPT)],
                    cntp_hbm.at[cid, pl.ds(sid * NPT, NPT)])


def _ew_deg_call(logits_p, q_p, row2d, col2d):
    kern = pl.kernel(
        _ew_deg_body,
        out_type=[
            jax.ShapeDtypeStruct((E_PAD // ECH, ECH), jnp.float32),
            jax.ShapeDtypeStruct((NC, N_PAD), jnp.float32),
            jax.ShapeDtypeStruct((NC, N_PAD), jnp.float32),
        ],
        mesh=_MESH,
        compiler_params=_SC_PARAMS,
        scratch_types=[
            pltpu.VMEM((NCH, ECH), jnp.int32),
            pltpu.VMEM((NCH, ECH), jnp.int32),
            pltpu.VMEM((2 * ECH, D_OUT), jnp.float32),
            pltpu.VMEM((2 * ECH, D_OUT), jnp.float32),
            pltpu.VMEM((2 * ECH,), jnp.float32),
            pltpu.VMEM((ECH,), jnp.float32),
            pltpu.VMEM((NPT,), jnp.float32),
            pltpu.VMEM_SHARED((N_PAD,), jnp.float32),
            pltpu.VMEM_SHARED((N_PAD,), jnp.float32),
            pltpu.SemaphoreType.DMA, pltpu.SemaphoreType.DMA,
            pltpu.SemaphoreType.DMA, pltpu.SemaphoreType.DMA,
            pltpu.SemaphoreType.DMA,
        ],
    )
    return kern(logits_p, q_p, row2d, col2d)


def _coef_body(ew_hbm, row_hbm, col_hbm, dis_hbm, a_hbm, b_hbm, coef_hbm,
               ridx, cidx, ewbuf, cfbuf, disv, avr, bvr):
    cid = lax.axis_index("c")
    sid = lax.axis_index("s")
    wid = cid * NS + sid
    base = wid * NCH

    pltpu.sync_copy(row_hbm.at[pl.ds(base, NCH)], ridx)
    pltpu.sync_copy(col_hbm.at[pl.ds(base, NCH)], cidx)
    pltpu.sync_copy(dis_hbm, disv)
    pltpu.sync_copy(ew_hbm.at[pl.ds(base, NCH)], ewbuf)
    pltpu.sync_copy(a_hbm, avr)
    pltpu.sync_copy(b_hbm, bvr)
    av = avr[...]
    bv = bvr[...]

    @pl.loop(0, NCH)
    def _(j):
        for g in range(ECH // LL):
            sl = pl.ds(g * LL, LL)
            rv = ridx[j, sl]
            cv = cidx[j, sl]
            dr = plsc.load_gather(disv, [rv])
            dc = plsc.load_gather(disv, [cv])
            cfbuf[j, sl] = (av * ewbuf[j, sl] + bv) * dr * dc

    pltpu.sync_copy(cfbuf, coef_hbm.at[pl.ds(base, NCH)])


def _coef_call(ew2d, row2d, col2d, dis_flat, a16, b16):
    kern = pl.kernel(
        _coef_body,
        out_type=jax.ShapeDtypeStruct((E_PAD // ECH, ECH), jnp.float32),
        mesh=_MESH,
        compiler_params=_SC_PARAMS,
        scratch_types=[
            pltpu.VMEM((NCH, ECH), jnp.int32),
            pltpu.VMEM((NCH, ECH), jnp.int32),
            pltpu.VMEM((NCH, ECH), jnp.float32),
            pltpu.VMEM((NCH, ECH), jnp.float32),
            pltpu.VMEM((N_PAD,), jnp.float32),
            pltpu.VMEM((LL,), jnp.float32),
            pltpu.VMEM((LL,), jnp.float32),
        ],
    )
    return kern(ew2d, row2d, col2d, dis_flat, a16, b16)


def _spmv_body(d, nb, nch, split, fuse, f_hbm, row_hbm, col_hbm, coef_hbm,
               *rest):
    if fuse:
        (alpha_hbm, beta_hbm, aggp_hbm,
         ridx, cidx, cvm, rows, abblk, agg_sh, *sems) = rest
    else:
        aggp_hbm, ridx, cidx, cvm, rows, agg_sh, *sems = rest
    gsems = sems[:nb]
    ssems = sems[nb:]
    pd = nb // 2
    cid = lax.axis_index("c")
    sid = lax.axis_index("s")
    if split:
        base = sid * nch
        table = f_hbm.at[cid]
    else:
        base = (cid * NS + sid) * nch
        table = f_hbm

    pltpu.sync_copy(row_hbm.at[pl.ds(base, nch)], ridx)
    pltpu.sync_copy(col_hbm.at[pl.ds(base, nch)], cidx)
    pltpu.sync_copy(coef_hbm.at[pl.ds(base, nch)], cvm)

    z = jnp.zeros((LL,), jnp.float32)

    @pl.loop(0, 64)
    def _(i):
        for jj in range(d // LL):
            rows[i, pl.ds(jj * LL, LL)] = z

    zsrc = rows.at[pl.ds(0, 64)]

    @pl.loop(0, NPT // 64)
    def _(k):
        pltpu.sync_copy(zsrc, agg_sh.at[pl.ds(sid * NPT + k * 64, 64)])

    plsc.subcore_barrier()

    def rows_at(b):
        return rows.at[pl.ds(b * ECH, ECH)]

    def issue_gather(k, b):
        pltpu.async_copy(table.at[cidx.at[k]], rows_at(b), gsems[b])

    def wait_gather(k, b):
        pltpu.make_async_copy(table.at[cidx.at[k]], rows_at(b), gsems[b]).wait()

    def issue_scatter(k, b):
        pltpu.async_copy(rows_at(b), agg_sh.at[ridx.at[k]], ssems[b], add=True)

    def wait_scatter(k, b):
        pltpu.make_async_copy(rows_at(b), agg_sh.at[ridx.at[k]],
                              ssems[b]).wait()

    def scale(k, b):
        kv = jnp.full((LL,), k, jnp.int32)

        @pl.loop(0, ECH, step=4)
        def _(e):
            r = rows_at(b)
            cbs = [plsc.load_gather(
                cvm, [kv, jnp.full((LL,), e + u, jnp.int32)])
                for u in range(4)]
            for u in range(4):
                for jj in range(d // LL):
                    sl = pl.ds(jj * LL, LL)
                    r[e + u, sl] = r[e + u, sl] * cbs[u]

    for k in range(pd):
        issue_gather(k, k)

    @pl.loop(0, nch // nb)
    def _(outer):
        k0 = outer * nb
        for i in range(nb):
            k = k0 + i
            wait_gather(k, i)
            bp = (i + pd) % nb
            kp = k + pd
            kold = kp - nb

            @pl.when(kp < nch)
            def _():
                @pl.when(kold >= 0)
                def _():
                    wait_scatter(kold, bp)
                issue_gather(kp, bp)

            scale(k, i)
            issue_scatter(k, i)

    for k in range(nch - nb, nch):
        wait_scatter(k, k % nb)

    plsc.subcore_barrier()
    if not fuse:
        pltpu.sync_copy(agg_sh.at[pl.ds(sid * NPT, NPT)],
                        aggp_hbm.at[cid, pl.ds(sid * NPT, NPT)])
    else:
        av = abblk.at[pl.ds(0, NPT)]
        bv = abblk.at[pl.ds(NPT, NPT)]
        pltpu.sync_copy(alpha_hbm.at[pl.ds(sid * NPT, NPT)], av)
        pltpu.sync_copy(beta_hbm.at[pl.ds(sid * NPT, NPT)], bv)
        sub = 64
        fblk = rows.at[pl.ds(0, sub)]
        ablk = rows.at[pl.ds(sub, sub)]

        @pl.loop(0, NPT // sub)
        def _(sb):
            r0 = sid * NPT + sb * sub
            pltpu.sync_copy(f_hbm.at[cid, pl.ds(r0, sub)], fblk)
            pltpu.sync_copy(agg_sh.at[pl.ds(r0, sub)], ablk)

            @pl.loop(0, sub, step=4)
            def _(rr):
                for u in range(4):
                    ri = sb * sub + rr + u
                    ab = plsc.load_gather(av, [jnp.full((LL,), ri, jnp.int32)])
                    bb = plsc.load_gather(bv, [jnp.full((LL,), ri, jnp.int32)])
                    for jj in range(d // LL):
                        sl = pl.ds(jj * LL, LL)
                        fblk[rr + u, sl] = (ab * ablk[rr + u, sl]
                                            + bb * fblk[rr + u, sl])

            pltpu.sync_copy(fblk, aggp_hbm.at[cid, pl.ds(r0, sub)])


@functools.cache
def _spmv_kernel(d, fuse=False):
    split = d == 64
    nb = 4
    nch = NCH * NC if split else NCH
    sems = [pltpu.SemaphoreType.DMA] * (2 * nb)
    out_t = jax.ShapeDtypeStruct((NC, N_PAD, d), jnp.float32)
    scratch = [
        pltpu.VMEM((nch, ECH), jnp.int32),
        pltpu.VMEM((nch, ECH), jnp.int32),
        pltpu.VMEM((nch, ECH), jnp.float32),
        pltpu.VMEM((nb * ECH, d), jnp.float32),
    ]
    if fuse:
        scratch.append(pltpu.VMEM((2 * NPT,), jnp.float32))
    scratch.append(pltpu.VMEM_SHARED((N_PAD, d), jnp.float32))
    return pl.kernel(
        functools.partial(_spmv_body, d, nb, nch, split, fuse),
        out_type=out_t,
        mesh=_MESH,
        compiler_params=_SC_PARAMS,
        scratch_types=scratch + sems,
    )


def _spmv_call(f, row2d, col2d, coef2d, d):
    return _spmv_kernel(d)(f, row2d, col2d, coef2d)


def _spmv_combine_call(f, row2d, col2d, coef2d, alpha_f, beta_f):
    return _spmv_kernel(64, True)(f, row2d, col2d, coef2d, alpha_f, beta_f)


def kernel(x, edge_index, W1, b1, Wp1, bp1, Wp2, bp2, Wp3, bp3, parsing0,
           Wc0, bc0, Wc1, bc1):
    x_p = jnp.pad(x, ((0, N_PAD - N), (0, 0)))
    row = edge_index[0]
    col = edge_index[1]
    row_p = jnp.concatenate(
        [row, jnp.full((E_PAD - E,), N, jnp.int32)]).reshape(E_PAD // ECH, ECH)
    col_p = jnp.concatenate(
        [col, jnp.zeros((E_PAD - E,), jnp.int32)]).reshape(E_PAD // ECH, ECH)

    logits_p, q_p, f0h = _front_call(
        x_p, Wp1, bp1, Wp2, bp2, Wp3, bp3, parsing0, W1, b1, Wc0, bc0)

    ew2d, degp, cntp = _ew_deg_call(logits_p, q_p, row_p, col_p)

    ew_valid = ew2d.reshape(-1)[:E].reshape(E // 128, 128)
    ab, dis, alpha, beta = _stats_fin_call(
        ew_valid,
        degp.reshape(NC, N_PAD // 128, 128),
        cntp.reshape(NC, N_PAD // 128, 128))

    dis_flat = dis.reshape(N_PAD)
    a16 = jnp.broadcast_to(ab.reshape(128)[0], (LL,))
    b16 = jnp.broadcast_to(ab.reshape(128)[1], (LL,))
    coef2d = _coef_call(ew2d, row_p, col_p, dis_flat, a16, b16)

    alpha_c = alpha.reshape(N_PAD, 1)
    beta_c = beta.reshape(N_PAD, 1)

    f1h = _spmv_combine_call(f0h, row_p, col_p, coef2d,
                             alpha.reshape(N_PAD), beta.reshape(N_PAD))
    p = _spmv_call(f1h, row_p, col_p, coef2d, 64)
    g0 = _comb_mm_call(p, f0h, alpha_c, beta_c, Wc1, bc1)

    p = _spmv_call(g0, row_p, col_p, coef2d, D_OUT)
    g1 = _combine_call(p[0], p[1], g0, alpha_c, beta_c, D_OUT)
    p = _spmv_call(g1, row_p, col_p, coef2d, D_OUT)
    g2 = _combine_call(p[0], p[1], g0, alpha_c, beta_c, D_OUT)

    return g2[:N]
